# baseline (speedup 1.0000x reference)
import functools

import jax
import jax.numpy as jnp
from jax import lax
from jax.experimental import pallas as pl
from jax.experimental.pallas import tpu as pltpu
from jax.experimental.pallas import tpu_sc as plsc

_N = 10000
_E = 160000
_H = 128
_MP = 3

_NC = 2
_NS = 16
_NW = _NC * _NS

_CH = 128
_EP = 163840
_JPW = _EP // (_NW * _CH)
_NPAD = 10016
_NPT = 624
_NREM = _N - _NS * _NPT

_NB = 1000
_EB = 1024


@functools.cache
def _sc_mesh():
    return plsc.VectorSubcoreMesh(
        core_axis_name="c", subcore_axis_name="s",
        num_cores=_NC, num_subcores=_NS)


def _f32dot(a, b):
    return jnp.dot(a.astype(jnp.bfloat16), b.astype(jnp.bfloat16),
                   preferred_element_type=jnp.float32)


def _ln(h, g, b):
    mu = jnp.mean(h, axis=-1, keepdims=True)
    var = jnp.mean((h - mu) ** 2, axis=-1, keepdims=True)
    return (h - mu) / jnp.sqrt(var + 1e-5) * g + b


def _fullspec(shape):
    n = len(shape)
    return pl.BlockSpec(shape, lambda i, _n=n: (0,) * _n)


def _rowspec(blk, d):
    return pl.BlockSpec((blk, d), lambda i: (i, 0))


def _enc_node_body(x_ref, w1_ref, b1_ref, w2_ref, b2_ref, w3_ref, b3_ref,
                   g_ref, be_ref, ws_ref, wd_ref, o_ref, p_ref, q_ref):
    x = x_ref[...]
    t = x[:, 0:1].astype(jnp.int32)
    oh = (lax.broadcasted_iota(jnp.int32, (_NB, 9), 1) == t).astype(jnp.float32)
    feats = jnp.concatenate([x[:, 1:3], oh], axis=-1)
    h = jnp.maximum(_f32dot(feats, w1_ref[...]) + b1_ref[...], 0.0)
    h = jnp.maximum(_f32dot(h, w2_ref[...]) + b2_ref[...], 0.0)
    h = _f32dot(h, w3_ref[...]) + b3_ref[...]
    hv = _ln(h, g_ref[...], be_ref[...])
    o_ref[...] = hv
    p_ref[...] = _f32dot(hv, ws_ref[...])
    q_ref[...] = _f32dot(hv, wd_ref[...])


def _enc_edge_body(x_ref, w1_ref, b1_ref, w2_ref, b2_ref, w3_ref, b3_ref,
                   g_ref, be_ref, o_ref):
    h = jnp.maximum(_f32dot(x_ref[...], w1_ref[...]) + b1_ref[...], 0.0)
    h = jnp.maximum(_f32dot(h, w2_ref[...]) + b2_ref[...], 0.0)
    h = _f32dot(h, w3_ref[...]) + b3_ref[...]
    o_ref[...] = _ln(h, g_ref[...], be_ref[...])


def _enc_edge_mlp_body(x_ref, a1_ref, c1_ref, a2_ref, c2_ref, a3_ref, c3_ref,
                       ag_ref, abe_ref, r_ref, w1_ref, b1_ref, w2_ref, b2_ref,
                       w3_ref, b3_ref, g_ref, be_ref, o_ref):
    h = jnp.maximum(_f32dot(x_ref[...], a1_ref[...]) + c1_ref[...], 0.0)
    h = jnp.maximum(_f32dot(h, a2_ref[...]) + c2_ref[...], 0.0)
    h = _f32dot(h, a3_ref[...]) + c3_ref[...]
    he = _ln(h, ag_ref[...], abe_ref[...])
    h = jnp.maximum(_f32dot(he, w1_ref[...]) + r_ref[...] + b1_ref[...], 0.0)
    h = jnp.maximum(_f32dot(h, w2_ref[...]) + b2_ref[...], 0.0)
    h = _f32dot(h, w3_ref[...]) + b3_ref[...]
    o_ref[...] = _ln(h, g_ref[...], be_ref[...]) + he


def _edge_mlp_body(he_ref, r_ref, w1_ref, b1_ref, w2_ref, b2_ref,
                   w3_ref, b3_ref, g_ref, be_ref, o_ref):
    he = he_ref[...]
    h = jnp.maximum(_f32dot(he, w1_ref[...]) + r_ref[...] + b1_ref[...], 0.0)
    h = jnp.maximum(_f32dot(h, w2_ref[...]) + b2_ref[...], 0.0)
    h = _f32dot(h, w3_ref[...]) + b3_ref[...]
    o_ref[...] = _ln(h, g_ref[...], be_ref[...]) + he


def _node_mlp_pq_body(hv_ref, part_ref, wv_ref, wa_ref, b1_ref, w2_ref,
                      b2_ref, w3_ref, b3_ref, g_ref, be_ref, ws_ref, wd_ref,
                      o_ref, p_ref, q_ref):
    hv = hv_ref[...]
    agg = part_ref[0] + part_ref[1]
    h = jnp.maximum(_f32dot(hv, wv_ref[...]) + _f32dot(agg, wa_ref[...])
                    + b1_ref[...], 0.0)
    h = jnp.maximum(_f32dot(h, w2_ref[...]) + b2_ref[...], 0.0)
    h = _f32dot(h, w3_ref[...]) + b3_ref[...]
    hv = _ln(h, g_ref[...], be_ref[...]) + hv
    o_ref[...] = hv
    p_ref[...] = _f32dot(hv, ws_ref[...])
    q_ref[...] = _f32dot(hv, wd_ref[...])


def _node_mlp_dec_body(hv_ref, part_ref, wv_ref, wa_ref, b1_ref, w2_ref,
                       b2_ref, w3_ref, b3_ref, g_ref, be_ref, fr_ref,
                       d1_ref, e1_ref, d2_ref, e2_ref, d3_ref, e3_ref,
                       o_ref):
    hv = hv_ref[...]
    agg = part_ref[0] + part_ref[1]
    h = jnp.maximum(_f32dot(hv, wv_ref[...]) + _f32dot(agg, wa_ref[...])
                    + b1_ref[...], 0.0)
    h = jnp.maximum(_f32dot(h, w2_ref[...]) + b2_ref[...], 0.0)
    h = _f32dot(h, w3_ref[...]) + b3_ref[...]
    hv = _ln(h, g_ref[...], be_ref[...]) + hv
    h = jnp.maximum(_f32dot(hv, d1_ref[...]) + e1_ref[...], 0.0)
    h = jnp.maximum(_f32dot(h, d2_ref[...]) + e2_ref[...], 0.0)
    o_ref[...] = fr_ref[...] + _f32dot(h, d3_ref[...]) + e3_ref[...]


def _rows_call(body, grid, in_arrays, in_blocked_d, out_shapes, out_d, blk):
    in_specs = []
    for a, d in zip(in_arrays, in_blocked_d):
        if d is None:
            in_specs.append(_fullspec(a.shape))
        elif isinstance(d, tuple):
            in_specs.append(pl.BlockSpec((d[0], blk, d[1]),
                                         lambda i: (0, i, 0)))
        else:
            in_specs.append(_rowspec(blk, d))
    out_specs = [_rowspec(blk, d) for d in out_d]
    out_shape = [jax.ShapeDtypeStruct(s, jnp.float32) for s in out_shapes]
    if len(out_shape) == 1:
        out_shape, out_specs = out_shape[0], out_specs[0]
    return pl.pallas_call(
        body, grid=(grid,), in_specs=in_specs, out_specs=out_specs,
        out_shape=out_shape)(*in_arrays)


def _sc_gather_body(p_hbm, q_hbm, src_hbm, dst_hbm, r_hbm,
                    sidx, didx, pbuf, qbuf,
                    isem0, isem1, gsem0, gsem1, wsem0, wsem1):
    cid = lax.axis_index("c")
    sid = lax.axis_index("s")
    wid = sid * _NC + cid
    isems = (isem0, isem1)
    gsems = (gsem0, gsem1)
    wsems = (wsem0, wsem1)

    def off(g):
        return (g * _NW + wid) * _CH

    def idx_start(g, s):
        pltpu.make_async_copy(src_hbm.at[pl.ds(off(g), _CH)],
                              sidx.at[s], isems[s]).start()
        pltpu.make_async_copy(dst_hbm.at[pl.ds(off(g), _CH)],
                              didx.at[s], isems[s]).start()

    def idx_wait(s):
        pltpu.make_async_copy(src_hbm.at[pl.ds(0, _CH)],
                              sidx.at[s], isems[s]).wait()
        pltpu.make_async_copy(dst_hbm.at[pl.ds(0, _CH)],
                              didx.at[s], isems[s]).wait()

    def gather_start(s):
        pltpu.make_async_copy(p_hbm.at[sidx.at[s]], pbuf.at[s],
                              gsems[s]).start()
        pltpu.make_async_copy(q_hbm.at[didx.at[s]], qbuf.at[s],
                              gsems[s]).start()

    def gather_wait(s):
        pltpu.make_async_copy(p_hbm.at[sidx.at[s]], pbuf.at[s],
                              gsems[s]).wait()
        pltpu.make_async_copy(q_hbm.at[didx.at[s]], qbuf.at[s],
                              gsems[s]).wait()

    def write_start(g, s):
        pltpu.make_async_copy(pbuf.at[s], r_hbm.at[pl.ds(off(g), _CH)],
                              wsems[s]).start()

    def write_wait(s):
        pltpu.make_async_copy(pbuf.at[s], r_hbm.at[pl.ds(0, _CH)],
                              wsems[s]).wait()

    def add_slot(s):
        def addrow(rr, c):
            for cc in range(_H // 16):
                col = pl.ds(cc * 16, 16)
                pbuf[s, rr, col] = pbuf[s, rr, col] + qbuf[s, rr, col]
            return c
        lax.fori_loop(0, _CH, addrow, 0)

    idx_start(0, 0)
    idx_wait(0)
    gather_start(0)
    idx_start(1, 1)

    def body(jj, carry):
        g0 = jj * 2
        gather_wait(0)

        @pl.when(jj < _JPW // 2 - 1)
        def _():
            idx_start(g0 + 2, 0)

        @pl.when(jj > 0)
        def _():
            write_wait(1)

        idx_wait(1)
        gather_start(1)
        add_slot(0)
        write_start(g0, 0)

        gather_wait(1)

        @pl.when(jj < _JPW // 2 - 1)
        def _():
            idx_start(g0 + 3, 1)

        write_wait(0)

        @pl.when(jj < _JPW // 2 - 1)
        def _():
            idx_wait(0)
            gather_start(0)

        add_slot(1)
        write_start(g0 + 1, 1)
        return carry

    lax.fori_loop(0, _JPW // 2, body, 0)
    write_wait(1)


@functools.cache
def _sc_gather_fn():
    return pl.kernel(
        _sc_gather_body,
        out_type=jax.ShapeDtypeStruct((_EP, _H), jnp.float32),
        mesh=_sc_mesh(),
        scratch_types=[
            pltpu.VMEM((2, _CH), jnp.int32),
            pltpu.VMEM((2, _CH), jnp.int32),
            pltpu.VMEM((2, _CH, _H), jnp.float32),
            pltpu.VMEM((2, _CH, _H), jnp.float32),
        ] + [pltpu.SemaphoreType.DMA] * 6)


def _sc_gather(p, q, src_g, dst_g):
    return _sc_gather_fn()(p, q, src_g, dst_g)


def _sc_scatter_body(e_hbm, dst_hbm, z_hbm, out_hbm,
                     didx, ebuf, acc, lsem0, lsem1, ssem0, ssem1):
    cid = lax.axis_index("c")
    sid = lax.axis_index("s")
    wid = sid * _NC + cid
    lsems = (lsem0, lsem1)
    ssems = (ssem0, ssem1)

    def off(g):
        return (g * _NW + wid) * _CH

    def load_start(g, s):
        pltpu.make_async_copy(dst_hbm.at[pl.ds(off(g), _CH)],
                              didx.at[s], lsems[s]).start()
        pltpu.make_async_copy(e_hbm.at[pl.ds(off(g), _CH)],
                              ebuf.at[s], lsems[s]).start()

    def load_wait(s):
        pltpu.make_async_copy(dst_hbm.at[pl.ds(0, _CH)],
                              didx.at[s], lsems[s]).wait()
        pltpu.make_async_copy(e_hbm.at[pl.ds(0, _CH)],
                              ebuf.at[s], lsems[s]).wait()

    def scat_start(s):
        pltpu.make_async_copy(ebuf.at[s], acc.at[didx.at[s]],
                              ssems[s]).start(add=True)

    def scat_wait(s):
        pltpu.make_async_copy(ebuf.at[s], acc.at[didx.at[s]],
                              ssems[s]).wait()

    pltpu.sync_copy(z_hbm.at[pl.ds(sid * _NPT, _NPT)],
                    acc.at[pl.ds(sid * _NPT, _NPT)])

    @pl.when(sid == _NS - 1)
    def _():
        pltpu.sync_copy(z_hbm.at[pl.ds(_NS * _NPT, _NREM)],
                        acc.at[pl.ds(_NS * _NPT, _NREM)])

    plsc.subcore_barrier()

    load_start(0, 0)

    def body(jj, carry):
        g0 = jj * 2
        load_wait(0)
        scat_start(0)

        @pl.when(jj > 0)
        def _():
            scat_wait(1)

        load_start(g0 + 1, 1)

        load_wait(1)
        scat_start(1)
        scat_wait(0)

        @pl.when(jj < _JPW // 2 - 1)
        def _():
            load_start(g0 + 2, 0)

        return carry

    lax.fori_loop(0, _JPW // 2, body, 0)
    scat_wait(1)
    plsc.subcore_barrier()
    pltpu.sync_copy(acc.at[pl.ds(sid * _NPT, _NPT)],
                    out_hbm.at[cid].at[pl.ds(sid * _NPT, _NPT)])

    @pl.when(sid == _NS - 1)
    def _():
        pltpu.sync_copy(acc.at[pl.ds(_NS * _NPT, _NREM)],
                        out_hbm.at[cid].at[pl.ds(_NS * _NPT, _NREM)])


@functools.cache
def _sc_scatter_fn():
    return pl.kernel(
        _sc_scatter_body,
        out_type=jax.ShapeDtypeStruct((_NC, _N, _H), jnp.float32),
        mesh=_sc_mesh(),
        scratch_types=[
            pltpu.VMEM((2, _CH), jnp.int32),
            pltpu.VMEM((2, _CH, _H), jnp.float32),
            pltpu.VMEM_SHARED((_NPAD, _H), jnp.float32),
        ] + [pltpu.SemaphoreType.DMA] * 4)


def _sc_scatter(e_new, dst_s, zeros_nh):
    return _sc_scatter_fn()(e_new, dst_s, zeros_nh)


def kernel(graph_x, edge_index, edge_attr, velocity_sequence_noise,
           enc_node, enc_edge, mp_edge, mp_node, dec, norm_stats):
    del velocity_sequence_noise
    node_mean, node_std, edge_mean, edge_std, out_mean, out_std = norm_stats
    f32 = jnp.float32
    r1 = lambda a: a.reshape(1, -1).astype(f32)

    nw1 = enc_node[0] / node_std[:, None]
    nb1 = r1(enc_node[1] - (node_mean / node_std) @ enc_node[0])
    ew1 = enc_edge[0] / edge_std[:, None]
    eb1 = r1(enc_edge[1] - (edge_mean / edge_std) @ enc_edge[0])

    npad = _EP - _E
    pad_iota = jnp.arange(npad, dtype=jnp.int32)
    src_g = jnp.concatenate(
        [edge_index[0].astype(jnp.int32), pad_iota % _N])
    dst = edge_index[1].astype(jnp.int32)
    dst_g = jnp.concatenate([dst, pad_iota % _N])
    dst_s = jnp.concatenate([dst, _N + pad_iota % (_NPAD - _N)])
    edge_attr_p = jnp.concatenate(
        [edge_attr, jnp.zeros((npad, edge_attr.shape[1]), f32)])
    frames = graph_x[:, 1:3]
    zeros_nh = jnp.zeros((_N, _H), f32)

    def edge_w(i):
        we1 = mp_edge[i][0]
        return we1[:_H], we1[_H:2 * _H], we1[2 * _H:]

    w1e0, w1s0, w1d0 = edge_w(0)
    h_v, p, q = _rows_call(
        _enc_node_body, _N // _NB,
        [graph_x, nw1, nb1, enc_node[2], r1(enc_node[3]), enc_node[4],
         r1(enc_node[5]), r1(enc_node[6]), r1(enc_node[7]), w1s0, w1d0],
        [3] + [None] * 10, [(_N, _H)] * 3, [_H] * 3, _NB)

    d1, db1, d2, db2, d3, db3 = dec
    d3f = d3 * out_std[None, :]
    db3f = r1(db3 * out_std + out_mean)

    h_e = None
    out = None
    for i in range(_MP):
        w1e, _, _ = edge_w(i)
        _, wb1, we2, wb2, we3, wb3, wg, wbe = mp_edge[i]

        r = _sc_gather(p, q, src_g, dst_g)

        if i == 0:
            e_new = _rows_call(
                _enc_edge_mlp_body, _EP // _EB,
                [edge_attr_p, ew1, eb1, enc_edge[2], r1(enc_edge[3]),
                 enc_edge[4], r1(enc_edge[5]), r1(enc_edge[6]),
                 r1(enc_edge[7]), r, w1e, r1(wb1), we2, r1(wb2), we3,
                 r1(wb3), r1(wg), r1(wbe)],
                [3] + [None] * 8 + [_H] + [None] * 8, [(_EP, _H)], [_H], _EB)
        else:
            e_new = _rows_call(
                _edge_mlp_body, _EP // _EB,
                [h_e, r, w1e, r1(wb1), we2, r1(wb2), we3, r1(wb3), r1(wg),
                 r1(wbe)],
                [_H, _H] + [None] * 8, [(_EP, _H)], [_H], _EB)

        part = _sc_scatter(e_new, dst_s, zeros_nh)

        wn1, nb1_, wn2, nb2_, wn3, nb3_, ng_, nbe_ = mp_node[i]
        wv, wa = wn1[:_H], wn1[_H:]
        if i < _MP - 1:
            _, w1sn, w1dn = edge_w(i + 1)
            h_v, p, q = _rows_call(
                _node_mlp_pq_body, _N // _NB,
                [h_v, part, wv, wa, r1(nb1_), wn2, r1(nb2_), wn3, r1(nb3_),
                 r1(ng_), r1(nbe_), w1sn, w1dn],
                [_H, (_NC, _H)] + [None] * 11, [(_N, _H)] * 3, [_H] * 3, _NB)
        else:
            out = _rows_call(
                _node_mlp_dec_body, _N // _NB,
                [h_v, part, wv, wa, r1(nb1_), wn2, r1(nb2_), wn3, r1(nb3_),
                 r1(ng_), r1(nbe_), frames, d1, r1(db1), d2, r1(db2), d3f,
                 db3f],
                [_H, (_NC, _H)] + [None] * 9 + [2] + [None] * 6,
                [(_N, 2)], [2], _NB)
        h_e = e_new
    return out


# speedup vs baseline: 1.1934x; 1.1934x over previous
import functools

import jax
import jax.numpy as jnp
from jax import lax
from jax.experimental import pallas as pl
from jax.experimental.pallas import tpu as pltpu
from jax.experimental.pallas import tpu_sc as plsc

_N = 10000
_E = 160000
_H = 128
_MP = 3

_NC = 2
_NS = 16
_NW = _NC * _NS

_CH = 128
_EP = 163840
_JPW = _EP // (_NW * _CH)
_NPAD = 10016
_NPT = 624
_NREM = _N - _NS * _NPT

_NB = 1000
_EB = 2048


@functools.cache
def _sc_mesh():
    return plsc.VectorSubcoreMesh(
        core_axis_name="c", subcore_axis_name="s",
        num_cores=_NC, num_subcores=_NS)


def _f32dot(a, b):
    return jnp.dot(a, b, preferred_element_type=jnp.float32)


def _ln(h, g, b):
    mu = jnp.mean(h, axis=-1, keepdims=True)
    d = h - mu
    var = jnp.mean(d * d, axis=-1, keepdims=True)
    return d * lax.rsqrt(var + 1e-5) * g + b


def _fullspec(shape):
    n = len(shape)
    return pl.BlockSpec(shape, lambda i, _n=n: (0,) * _n)


def _rowspec(blk, d):
    return pl.BlockSpec((blk, d), lambda i: (i, 0))


def _enc_node_body(x_ref, w1_ref, b1_ref, w2_ref, b2_ref, w3_ref, b3_ref,
                   g_ref, be_ref, ws_ref, wd_ref, o_ref, p_ref, q_ref):
    x = x_ref[...]
    t = x[:, 0:1].astype(jnp.int32)
    oh = (lax.broadcasted_iota(jnp.int32, (_NB, 9), 1) == t).astype(jnp.float32)
    feats = jnp.concatenate([x[:, 1:3], oh], axis=-1)
    h = jnp.maximum(_f32dot(feats, w1_ref[...]) + b1_ref[...], 0.0)
    h = jnp.maximum(_f32dot(h, w2_ref[...]) + b2_ref[...], 0.0)
    h = _f32dot(h, w3_ref[...]) + b3_ref[...]
    hv = _ln(h, g_ref[...], be_ref[...])
    o_ref[...] = hv
    p_ref[...] = _f32dot(hv, ws_ref[...])
    q_ref[...] = _f32dot(hv, wd_ref[...])


def _enc_edge_body(x_ref, w1_ref, b1_ref, w2_ref, b2_ref, w3_ref, b3_ref,
                   g_ref, be_ref, o_ref):
    h = jnp.maximum(_f32dot(x_ref[...], w1_ref[...]) + b1_ref[...], 0.0)
    h = jnp.maximum(_f32dot(h, w2_ref[...]) + b2_ref[...], 0.0)
    h = _f32dot(h, w3_ref[...]) + b3_ref[...]
    o_ref[...] = _ln(h, g_ref[...], be_ref[...])


def _enc_edge_mlp_body(x_ref, a1_ref, c1_ref, a2_ref, c2_ref, a3_ref, c3_ref,
                       ag_ref, abe_ref, r_ref, w1_ref, b1_ref, w2_ref, b2_ref,
                       w3_ref, b3_ref, g_ref, be_ref, o_ref):
    h = jnp.maximum(_f32dot(x_ref[...], a1_ref[...]) + c1_ref[...], 0.0)
    h = jnp.maximum(_f32dot(h, a2_ref[...]) + c2_ref[...], 0.0)
    h = _f32dot(h, a3_ref[...]) + c3_ref[...]
    he = _ln(h, ag_ref[...], abe_ref[...])
    h = jnp.maximum(_f32dot(he, w1_ref[...]) + r_ref[...] + b1_ref[...], 0.0)
    h = jnp.maximum(_f32dot(h, w2_ref[...]) + b2_ref[...], 0.0)
    h = _f32dot(h, w3_ref[...]) + b3_ref[...]
    o_ref[...] = _ln(h, g_ref[...], be_ref[...]) + he


def _edge_mlp_body(he_ref, r_ref, w1_ref, b1_ref, w2_ref, b2_ref,
                   w3_ref, b3_ref, g_ref, be_ref, o_ref):
    he = he_ref[...]
    h = jnp.maximum(_f32dot(he, w1_ref[...]) + r_ref[...] + b1_ref[...], 0.0)
    h = jnp.maximum(_f32dot(h, w2_ref[...]) + b2_ref[...], 0.0)
    h = _f32dot(h, w3_ref[...]) + b3_ref[...]
    o_ref[...] = _ln(h, g_ref[...], be_ref[...]) + he


def _node_mlp_pq_body(hv_ref, part_ref, wv_ref, wa_ref, b1_ref, w2_ref,
                      b2_ref, w3_ref, b3_ref, g_ref, be_ref, ws_ref, wd_ref,
                      o_ref, p_ref, q_ref):
    hv = hv_ref[...]
    agg = part_ref[0] + part_ref[1]
    h = jnp.maximum(_f32dot(hv, wv_ref[...]) + _f32dot(agg, wa_ref[...])
                    + b1_ref[...], 0.0)
    h = jnp.maximum(_f32dot(h, w2_ref[...]) + b2_ref[...], 0.0)
    h = _f32dot(h, w3_ref[...]) + b3_ref[...]
    hv = _ln(h, g_ref[...], be_ref[...]) + hv
    o_ref[...] = hv
    p_ref[...] = _f32dot(hv, ws_ref[...])
    q_ref[...] = _f32dot(hv, wd_ref[...])


def _node_mlp_dec_body(hv_ref, part_ref, wv_ref, wa_ref, b1_ref, w2_ref,
                       b2_ref, w3_ref, b3_ref, g_ref, be_ref, fr_ref,
                       d1_ref, e1_ref, d2_ref, e2_ref, d3_ref, e3_ref,
                       o_ref):
    hv = hv_ref[...]
    agg = part_ref[0] + part_ref[1]
    h = jnp.maximum(_f32dot(hv, wv_ref[...]) + _f32dot(agg, wa_ref[...])
                    + b1_ref[...], 0.0)
    h = jnp.maximum(_f32dot(h, w2_ref[...]) + b2_ref[...], 0.0)
    h = _f32dot(h, w3_ref[...]) + b3_ref[...]
    hv = _ln(h, g_ref[...], be_ref[...]) + hv
    h = jnp.maximum(_f32dot(hv, d1_ref[...]) + e1_ref[...], 0.0)
    h = jnp.maximum(_f32dot(h, d2_ref[...]) + e2_ref[...], 0.0)
    o_ref[...] = fr_ref[...] + _f32dot(h, d3_ref[...]) + e3_ref[...]


def _rows_call(body, grid, in_arrays, in_blocked_d, out_shapes, out_d, blk):
    in_specs = []
    for a, d in zip(in_arrays, in_blocked_d):
        if d is None:
            in_specs.append(_fullspec(a.shape))
        elif isinstance(d, tuple):
            in_specs.append(pl.BlockSpec((d[0], blk, d[1]),
                                         lambda i: (0, i, 0)))
        else:
            in_specs.append(_rowspec(blk, d))
    out_specs = [_rowspec(blk, d) for d in out_d]
    out_shape = [jax.ShapeDtypeStruct(s, jnp.float32) for s in out_shapes]
    if len(out_shape) == 1:
        out_shape, out_specs = out_shape[0], out_specs[0]
    return pl.pallas_call(
        body, grid=(grid,), in_specs=in_specs, out_specs=out_specs,
        out_shape=out_shape)(*in_arrays)


def _sc_gather_body(p_hbm, q_hbm, src_hbm, dst_hbm, r_hbm,
                    sidx, didx, pbuf, qbuf,
                    isem0, isem1, gsem0, gsem1, wsem0, wsem1):
    cid = lax.axis_index("c")
    sid = lax.axis_index("s")
    wid = sid * _NC + cid
    isems = (isem0, isem1)
    gsems = (gsem0, gsem1)
    wsems = (wsem0, wsem1)

    def off(g):
        return (g * _NW + wid) * _CH

    def idx_start(g, s):
        pltpu.make_async_copy(src_hbm.at[pl.ds(off(g), _CH)],
                              sidx.at[s], isems[s]).start()
        pltpu.make_async_copy(dst_hbm.at[pl.ds(off(g), _CH)],
                              didx.at[s], isems[s]).start()

    def idx_wait(s):
        pltpu.make_async_copy(src_hbm.at[pl.ds(0, _CH)],
                              sidx.at[s], isems[s]).wait()
        pltpu.make_async_copy(dst_hbm.at[pl.ds(0, _CH)],
                              didx.at[s], isems[s]).wait()

    def gather_start(s):
        pltpu.make_async_copy(p_hbm.at[sidx.at[s]], pbuf.at[s],
                              gsems[s]).start()
        pltpu.make_async_copy(q_hbm.at[didx.at[s]], qbuf.at[s],
                              gsems[s]).start()

    def gather_wait(s):
        pltpu.make_async_copy(p_hbm.at[sidx.at[s]], pbuf.at[s],
                              gsems[s]).wait()
        pltpu.make_async_copy(q_hbm.at[didx.at[s]], qbuf.at[s],
                              gsems[s]).wait()

    def write_start(g, s):
        pltpu.make_async_copy(pbuf.at[s], r_hbm.at[pl.ds(off(g), _CH)],
                              wsems[s]).start()

    def write_wait(s):
        pltpu.make_async_copy(pbuf.at[s], r_hbm.at[pl.ds(0, _CH)],
                              wsems[s]).wait()

    def add_slot(s):
        def addrow(rr, c):
            for cc in range(_H // 16):
                col = pl.ds(cc * 16, 16)
                pbuf[s, rr, col] = pbuf[s, rr, col] + qbuf[s, rr, col]
            return c
        lax.fori_loop(0, _CH, addrow, 0)

    idx_start(0, 0)
    idx_wait(0)
    gather_start(0)
    idx_start(1, 1)

    def body(jj, carry):
        g0 = jj * 2
        gather_wait(0)

        @pl.when(jj < _JPW // 2 - 1)
        def _():
            idx_start(g0 + 2, 0)

        @pl.when(jj > 0)
        def _():
            write_wait(1)

        idx_wait(1)
        gather_start(1)
        add_slot(0)
        write_start(g0, 0)

        gather_wait(1)

        @pl.when(jj < _JPW // 2 - 1)
        def _():
            idx_start(g0 + 3, 1)

        write_wait(0)

        @pl.when(jj < _JPW // 2 - 1)
        def _():
            idx_wait(0)
            gather_start(0)

        add_slot(1)
        write_start(g0 + 1, 1)
        return carry

    lax.fori_loop(0, _JPW // 2, body, 0)
    write_wait(1)


@functools.cache
def _sc_gather_fn():
    return pl.kernel(
        _sc_gather_body,
        out_type=jax.ShapeDtypeStruct((_EP, _H), jnp.float32),
        mesh=_sc_mesh(),
        scratch_types=[
            pltpu.VMEM((2, _CH), jnp.int32),
            pltpu.VMEM((2, _CH), jnp.int32),
            pltpu.VMEM((2, _CH, _H), jnp.float32),
            pltpu.VMEM((2, _CH, _H), jnp.float32),
        ] + [pltpu.SemaphoreType.DMA] * 6)


def _sc_gather(p, q, src_g, dst_g):
    return _sc_gather_fn()(p, q, src_g, dst_g)


def _sc_scatter_body(e_hbm, dst_hbm, z_hbm, out_hbm,
                     didx, ebuf, acc, lsem0, lsem1, ssem0, ssem1):
    cid = lax.axis_index("c")
    sid = lax.axis_index("s")
    wid = sid * _NC + cid
    lsems = (lsem0, lsem1)
    ssems = (ssem0, ssem1)

    def off(g):
        return (g * _NW + wid) * _CH

    def load_start(g, s):
        pltpu.make_async_copy(dst_hbm.at[pl.ds(off(g), _CH)],
                              didx.at[s], lsems[s]).start()
        pltpu.make_async_copy(e_hbm.at[pl.ds(off(g), _CH)],
                              ebuf.at[s], lsems[s]).start()

    def load_wait(s):
        pltpu.make_async_copy(dst_hbm.at[pl.ds(0, _CH)],
                              didx.at[s], lsems[s]).wait()
        pltpu.make_async_copy(e_hbm.at[pl.ds(0, _CH)],
                              ebuf.at[s], lsems[s]).wait()

    def scat_start(s):
        pltpu.make_async_copy(ebuf.at[s], acc.at[didx.at[s]],
                              ssems[s]).start(add=True)

    def scat_wait(s):
        pltpu.make_async_copy(ebuf.at[s], acc.at[didx.at[s]],
                              ssems[s]).wait()

    pltpu.sync_copy(z_hbm.at[pl.ds(sid * _NPT, _NPT)],
                    acc.at[pl.ds(sid * _NPT, _NPT)])

    @pl.when(sid == _NS - 1)
    def _():
        pltpu.sync_copy(z_hbm.at[pl.ds(_NS * _NPT, _NREM)],
                        acc.at[pl.ds(_NS * _NPT, _NREM)])

    plsc.subcore_barrier()

    load_start(0, 0)

    def body(jj, carry):
        g0 = jj * 2
        load_wait(0)
        scat_start(0)

        @pl.when(jj > 0)
        def _():
            scat_wait(1)

        load_start(g0 + 1, 1)

        load_wait(1)
        scat_start(1)
        scat_wait(0)

        @pl.when(jj < _JPW // 2 - 1)
        def _():
            load_start(g0 + 2, 0)

        return carry

    lax.fori_loop(0, _JPW // 2, body, 0)
    scat_wait(1)
    plsc.subcore_barrier()
    pltpu.sync_copy(acc.at[pl.ds(sid * _NPT, _NPT)],
                    out_hbm.at[cid].at[pl.ds(sid * _NPT, _NPT)])

    @pl.when(sid == _NS - 1)
    def _():
        pltpu.sync_copy(acc.at[pl.ds(_NS * _NPT, _NREM)],
                        out_hbm.at[cid].at[pl.ds(_NS * _NPT, _NREM)])


@functools.cache
def _sc_scatter_fn():
    return pl.kernel(
        _sc_scatter_body,
        out_type=jax.ShapeDtypeStruct((_NC, _N, _H), jnp.float32),
        mesh=_sc_mesh(),
        scratch_types=[
            pltpu.VMEM((2, _CH), jnp.int32),
            pltpu.VMEM((2, _CH, _H), jnp.float32),
            pltpu.VMEM_SHARED((_NPAD, _H), jnp.float32),
        ] + [pltpu.SemaphoreType.DMA] * 4)


def _sc_scatter(e_new, dst_s, zeros_nh):
    return _sc_scatter_fn()(e_new, dst_s, zeros_nh)


def kernel(graph_x, edge_index, edge_attr, velocity_sequence_noise,
           enc_node, enc_edge, mp_edge, mp_node, dec, norm_stats):
    del velocity_sequence_noise
    node_mean, node_std, edge_mean, edge_std, out_mean, out_std = norm_stats
    f32 = jnp.float32
    r1 = lambda a: a.reshape(1, -1).astype(f32)

    nw1 = enc_node[0] / node_std[:, None]
    nb1 = r1(enc_node[1] - (node_mean / node_std) @ enc_node[0])
    ew1 = enc_edge[0] / edge_std[:, None]
    eb1 = r1(enc_edge[1] - (edge_mean / edge_std) @ enc_edge[0])

    npad = _EP - _E
    pad_iota = jnp.arange(npad, dtype=jnp.int32)
    src_g = jnp.concatenate(
        [edge_index[0].astype(jnp.int32), pad_iota % _N])
    dst = edge_index[1].astype(jnp.int32)
    dst_g = jnp.concatenate([dst, pad_iota % _N])
    dst_s = jnp.concatenate([dst, _N + pad_iota % (_NPAD - _N)])
    edge_attr_p = jnp.concatenate(
        [edge_attr, jnp.zeros((npad, edge_attr.shape[1]), f32)])
    frames = graph_x[:, 1:3]
    zeros_nh = jnp.zeros((_N, _H), f32)

    def edge_w(i):
        we1 = mp_edge[i][0]
        return we1[:_H], we1[_H:2 * _H], we1[2 * _H:]

    w1e0, w1s0, w1d0 = edge_w(0)
    h_v, p, q = _rows_call(
        _enc_node_body, _N // _NB,
        [graph_x, nw1, nb1, enc_node[2], r1(enc_node[3]), enc_node[4],
         r1(enc_node[5]), r1(enc_node[6]), r1(enc_node[7]), w1s0, w1d0],
        [3] + [None] * 10, [(_N, _H)] * 3, [_H] * 3, _NB)

    d1, db1, d2, db2, d3, db3 = dec
    d3f = d3 * out_std[None, :]
    db3f = r1(db3 * out_std + out_mean)

    h_e = None
    out = None
    for i in range(_MP):
        w1e, _, _ = edge_w(i)
        _, wb1, we2, wb2, we3, wb3, wg, wbe = mp_edge[i]

        r = _sc_gather(p, q, src_g, dst_g)

        if i == 0:
            e_new = _rows_call(
                _enc_edge_mlp_body, _EP // _EB,
                [edge_attr_p, ew1, eb1, enc_edge[2], r1(enc_edge[3]),
                 enc_edge[4], r1(enc_edge[5]), r1(enc_edge[6]),
                 r1(enc_edge[7]), r, w1e, r1(wb1), we2, r1(wb2), we3,
                 r1(wb3), r1(wg), r1(wbe)],
                [3] + [None] * 8 + [_H] + [None] * 8, [(_EP, _H)], [_H], _EB)
        else:
            e_new = _rows_call(
                _edge_mlp_body, _EP // _EB,
                [h_e, r, w1e, r1(wb1), we2, r1(wb2), we3, r1(wb3), r1(wg),
                 r1(wbe)],
                [_H, _H] + [None] * 8, [(_EP, _H)], [_H], _EB)

        part = _sc_scatter(e_new, dst_s, zeros_nh)

        wn1, nb1_, wn2, nb2_, wn3, nb3_, ng_, nbe_ = mp_node[i]
        wv, wa = wn1[:_H], wn1[_H:]
        if i < _MP - 1:
            _, w1sn, w1dn = edge_w(i + 1)
            h_v, p, q = _rows_call(
                _node_mlp_pq_body, _N // _NB,
                [h_v, part, wv, wa, r1(nb1_), wn2, r1(nb2_), wn3, r1(nb3_),
                 r1(ng_), r1(nbe_), w1sn, w1dn],
                [_H, (_NC, _H)] + [None] * 11, [(_N, _H)] * 3, [_H] * 3, _NB)
        else:
            out = _rows_call(
                _node_mlp_dec_body, _N // _NB,
                [h_v, part, wv, wa, r1(nb1_), wn2, r1(nb2_), wn3, r1(nb3_),
                 r1(ng_), r1(nbe_), frames, d1, r1(db1), d2, r1(db2), d3f,
                 db3f],
                [_H, (_NC, _H)] + [None] * 9 + [2] + [None] * 6,
                [(_N, 2)], [2], _NB)
        h_e = e_new
    return out


# speedup vs baseline: 1.3200x; 1.1061x over previous
import functools

import jax
import jax.numpy as jnp
from jax import lax
from jax.experimental import pallas as pl
from jax.experimental.pallas import tpu as pltpu
from jax.experimental.pallas import tpu_sc as plsc

_N = 10000
_E = 160000
_H = 128
_MP = 3

_NC = 2
_NS = 16
_NW = _NC * _NS

_CH = 128
_EP = 163840
_JPW = _EP // (_NW * _CH)
_NPAD = 10016
_NPT = 624
_NREM = _N - _NS * _NPT

_NB = 2000
_EB = 4096


@functools.cache
def _sc_mesh():
    return plsc.VectorSubcoreMesh(
        core_axis_name="c", subcore_axis_name="s",
        num_cores=_NC, num_subcores=_NS)


def _f32dot(a, b):
    return jnp.dot(a, b, preferred_element_type=jnp.float32)


def _ln(h, g, b):
    mu = jnp.mean(h, axis=-1, keepdims=True)
    d = h - mu
    var = jnp.mean(d * d, axis=-1, keepdims=True)
    return d * lax.rsqrt(var + 1e-5) * g + b


def _fullspec(shape):
    n = len(shape)
    return pl.BlockSpec(shape, lambda i, _n=n: (0,) * _n)


def _rowspec(blk, d):
    return pl.BlockSpec((blk, d), lambda i: (i, 0))


def _enc_node_body(x_ref, w1_ref, b1_ref, w2_ref, b2_ref, w3_ref, b3_ref,
                   g_ref, be_ref, ws_ref, wd_ref, o_ref, p_ref, q_ref):
    x = x_ref[...]
    t = x[:, 0:1].astype(jnp.int32)
    oh = (lax.broadcasted_iota(jnp.int32, (_NB, 9), 1) == t).astype(jnp.float32)
    feats = jnp.concatenate([x[:, 1:3], oh], axis=-1)
    h = jnp.maximum(_f32dot(feats, w1_ref[...]) + b1_ref[...], 0.0)
    h = jnp.maximum(_f32dot(h, w2_ref[...]) + b2_ref[...], 0.0)
    h = _f32dot(h, w3_ref[...]) + b3_ref[...]
    hv = _ln(h, g_ref[...], be_ref[...])
    o_ref[...] = hv
    p_ref[...] = _f32dot(hv, ws_ref[...])
    q_ref[...] = _f32dot(hv, wd_ref[...])


def _enc_edge_body(x_ref, w1_ref, b1_ref, w2_ref, b2_ref, w3_ref, b3_ref,
                   g_ref, be_ref, o_ref):
    h = jnp.maximum(_f32dot(x_ref[...], w1_ref[...]) + b1_ref[...], 0.0)
    h = jnp.maximum(_f32dot(h, w2_ref[...]) + b2_ref[...], 0.0)
    h = _f32dot(h, w3_ref[...]) + b3_ref[...]
    o_ref[...] = _ln(h, g_ref[...], be_ref[...])


def _enc_edge_mlp_body(x_ref, a1_ref, c1_ref, a2_ref, c2_ref, a3_ref, c3_ref,
                       ag_ref, abe_ref, r_ref, w1_ref, b1_ref, w2_ref, b2_ref,
                       w3_ref, b3_ref, g_ref, be_ref, o_ref):
    h = jnp.maximum(_f32dot(x_ref[...], a1_ref[...]) + c1_ref[...], 0.0)
    h = jnp.maximum(_f32dot(h, a2_ref[...]) + c2_ref[...], 0.0)
    h = _f32dot(h, a3_ref[...]) + c3_ref[...]
    he = _ln(h, ag_ref[...], abe_ref[...])
    h = jnp.maximum(_f32dot(he, w1_ref[...]) + r_ref[...] + b1_ref[...], 0.0)
    h = jnp.maximum(_f32dot(h, w2_ref[...]) + b2_ref[...], 0.0)
    h = _f32dot(h, w3_ref[...]) + b3_ref[...]
    o_ref[...] = _ln(h, g_ref[...], be_ref[...]) + he


def _edge_mlp_body(he_ref, r_ref, w1_ref, b1_ref, w2_ref, b2_ref,
                   w3_ref, b3_ref, g_ref, be_ref, o_ref):
    he = he_ref[...]
    h = jnp.maximum(_f32dot(he, w1_ref[...]) + r_ref[...] + b1_ref[...], 0.0)
    h = jnp.maximum(_f32dot(h, w2_ref[...]) + b2_ref[...], 0.0)
    h = _f32dot(h, w3_ref[...]) + b3_ref[...]
    o_ref[...] = _ln(h, g_ref[...], be_ref[...]) + he


def _node_mlp_pq_body(hv_ref, part_ref, wv_ref, wa_ref, b1_ref, w2_ref,
                      b2_ref, w3_ref, b3_ref, g_ref, be_ref, ws_ref, wd_ref,
                      o_ref, p_ref, q_ref):
    hv = hv_ref[...]
    agg = part_ref[0] + part_ref[1]
    h = jnp.maximum(_f32dot(hv, wv_ref[...]) + _f32dot(agg, wa_ref[...])
                    + b1_ref[...], 0.0)
    h = jnp.maximum(_f32dot(h, w2_ref[...]) + b2_ref[...], 0.0)
    h = _f32dot(h, w3_ref[...]) + b3_ref[...]
    hv = _ln(h, g_ref[...], be_ref[...]) + hv
    o_ref[...] = hv
    p_ref[...] = _f32dot(hv, ws_ref[...])
    q_ref[...] = _f32dot(hv, wd_ref[...])


def _node_mlp_dec_body(hv_ref, part_ref, wv_ref, wa_ref, b1_ref, w2_ref,
                       b2_ref, w3_ref, b3_ref, g_ref, be_ref, fr_ref,
                       d1_ref, e1_ref, d2_ref, e2_ref, d3_ref, e3_ref,
                       o_ref):
    hv = hv_ref[...]
    agg = part_ref[0] + part_ref[1]
    h = jnp.maximum(_f32dot(hv, wv_ref[...]) + _f32dot(agg, wa_ref[...])
                    + b1_ref[...], 0.0)
    h = jnp.maximum(_f32dot(h, w2_ref[...]) + b2_ref[...], 0.0)
    h = _f32dot(h, w3_ref[...]) + b3_ref[...]
    hv = _ln(h, g_ref[...], be_ref[...]) + hv
    h = jnp.maximum(_f32dot(hv, d1_ref[...]) + e1_ref[...], 0.0)
    h = jnp.maximum(_f32dot(h, d2_ref[...]) + e2_ref[...], 0.0)
    o_ref[...] = fr_ref[...] + _f32dot(h, d3_ref[...]) + e3_ref[...]


def _rows_call(body, grid, in_arrays, in_blocked_d, out_shapes, out_d, blk):
    in_specs = []
    for a, d in zip(in_arrays, in_blocked_d):
        if d is None:
            in_specs.append(_fullspec(a.shape))
        elif isinstance(d, tuple):
            in_specs.append(pl.BlockSpec((d[0], blk, d[1]),
                                         lambda i: (0, i, 0)))
        else:
            in_specs.append(_rowspec(blk, d))
    out_specs = [_rowspec(blk, d) for d in out_d]
    out_shape = [jax.ShapeDtypeStruct(s, jnp.float32) for s in out_shapes]
    if len(out_shape) == 1:
        out_shape, out_specs = out_shape[0], out_specs[0]
    return pl.pallas_call(
        body, grid=(grid,), in_specs=in_specs, out_specs=out_specs,
        out_shape=out_shape)(*in_arrays)


def _sc_gather_body(p_hbm, q_hbm, src_hbm, dst_hbm, r_hbm,
                    sidx, didx, pbuf, qbuf,
                    isem0, isem1, gsem0, gsem1, wsem0, wsem1):
    cid = lax.axis_index("c")
    sid = lax.axis_index("s")
    wid = sid * _NC + cid
    isems = (isem0, isem1)
    gsems = (gsem0, gsem1)
    wsems = (wsem0, wsem1)

    def off(g):
        return (g * _NW + wid) * _CH

    def idx_start(g, s):
        pltpu.make_async_copy(src_hbm.at[pl.ds(off(g), _CH)],
                              sidx.at[s], isems[s]).start()
        pltpu.make_async_copy(dst_hbm.at[pl.ds(off(g), _CH)],
                              didx.at[s], isems[s]).start()

    def idx_wait(s):
        pltpu.make_async_copy(src_hbm.at[pl.ds(0, _CH)],
                              sidx.at[s], isems[s]).wait()
        pltpu.make_async_copy(dst_hbm.at[pl.ds(0, _CH)],
                              didx.at[s], isems[s]).wait()

    def gather_start(s):
        pltpu.make_async_copy(p_hbm.at[sidx.at[s]], pbuf.at[s],
                              gsems[s]).start()
        pltpu.make_async_copy(q_hbm.at[didx.at[s]], qbuf.at[s],
                              gsems[s]).start()

    def gather_wait(s):
        pltpu.make_async_copy(p_hbm.at[sidx.at[s]], pbuf.at[s],
                              gsems[s]).wait()
        pltpu.make_async_copy(q_hbm.at[didx.at[s]], qbuf.at[s],
                              gsems[s]).wait()

    def write_start(g, s):
        pltpu.make_async_copy(pbuf.at[s], r_hbm.at[pl.ds(off(g), _CH)],
                              wsems[s]).start()

    def write_wait(s):
        pltpu.make_async_copy(pbuf.at[s], r_hbm.at[pl.ds(0, _CH)],
                              wsems[s]).wait()

    def add_slot(s):
        def addrow(rr, c):
            for cc in range(_H // 16):
                col = pl.ds(cc * 16, 16)
                pbuf[s, rr, col] = pbuf[s, rr, col] + qbuf[s, rr, col]
            return c
        lax.fori_loop(0, _CH, addrow, 0)

    idx_start(0, 0)
    idx_wait(0)
    gather_start(0)
    idx_start(1, 1)

    def body(jj, carry):
        g0 = jj * 2
        gather_wait(0)

        @pl.when(jj < _JPW // 2 - 1)
        def _():
            idx_start(g0 + 2, 0)

        @pl.when(jj > 0)
        def _():
            write_wait(1)

        idx_wait(1)
        gather_start(1)
        add_slot(0)
        write_start(g0, 0)

        gather_wait(1)

        @pl.when(jj < _JPW // 2 - 1)
        def _():
            idx_start(g0 + 3, 1)

        write_wait(0)

        @pl.when(jj < _JPW // 2 - 1)
        def _():
            idx_wait(0)
            gather_start(0)

        add_slot(1)
        write_start(g0 + 1, 1)
        return carry

    lax.fori_loop(0, _JPW // 2, body, 0)
    write_wait(1)


@functools.cache
def _sc_gather_fn():
    return pl.kernel(
        _sc_gather_body,
        out_type=jax.ShapeDtypeStruct((_EP, _H), jnp.float32),
        mesh=_sc_mesh(),
        scratch_types=[
            pltpu.VMEM((2, _CH), jnp.int32),
            pltpu.VMEM((2, _CH), jnp.int32),
            pltpu.VMEM((2, _CH, _H), jnp.float32),
            pltpu.VMEM((2, _CH, _H), jnp.float32),
        ] + [pltpu.SemaphoreType.DMA] * 6)


def _sc_gather(p, q, src_g, dst_g):
    return _sc_gather_fn()(p, q, src_g, dst_g)


def _sc_scatter_body(e_hbm, dst_hbm, z_hbm, out_hbm,
                     didx, ebuf, acc, lsem0, lsem1, ssem0, ssem1):
    cid = lax.axis_index("c")
    sid = lax.axis_index("s")
    wid = sid * _NC + cid
    lsems = (lsem0, lsem1)
    ssems = (ssem0, ssem1)

    def off(g):
        return (g * _NW + wid) * _CH

    def load_start(g, s):
        pltpu.make_async_copy(dst_hbm.at[pl.ds(off(g), _CH)],
                              didx.at[s], lsems[s]).start()
        pltpu.make_async_copy(e_hbm.at[pl.ds(off(g), _CH)],
                              ebuf.at[s], lsems[s]).start()

    def load_wait(s):
        pltpu.make_async_copy(dst_hbm.at[pl.ds(0, _CH)],
                              didx.at[s], lsems[s]).wait()
        pltpu.make_async_copy(e_hbm.at[pl.ds(0, _CH)],
                              ebuf.at[s], lsems[s]).wait()

    def scat_start(s):
        pltpu.make_async_copy(ebuf.at[s], acc.at[didx.at[s]],
                              ssems[s]).start(add=True)

    def scat_wait(s):
        pltpu.make_async_copy(ebuf.at[s], acc.at[didx.at[s]],
                              ssems[s]).wait()

    pltpu.sync_copy(z_hbm.at[pl.ds(sid * _NPT, _NPT)],
                    acc.at[pl.ds(sid * _NPT, _NPT)])

    @pl.when(sid == _NS - 1)
    def _():
        pltpu.sync_copy(z_hbm.at[pl.ds(_NS * _NPT, _NREM)],
                        acc.at[pl.ds(_NS * _NPT, _NREM)])

    plsc.subcore_barrier()

    load_start(0, 0)

    def body(jj, carry):
        g0 = jj * 2
        load_wait(0)
        scat_start(0)

        @pl.when(jj > 0)
        def _():
            scat_wait(1)

        load_start(g0 + 1, 1)

        load_wait(1)
        scat_start(1)
        scat_wait(0)

        @pl.when(jj < _JPW // 2 - 1)
        def _():
            load_start(g0 + 2, 0)

        return carry

    lax.fori_loop(0, _JPW // 2, body, 0)
    scat_wait(1)
    plsc.subcore_barrier()
    pltpu.sync_copy(acc.at[pl.ds(sid * _NPT, _NPT)],
                    out_hbm.at[cid].at[pl.ds(sid * _NPT, _NPT)])

    @pl.when(sid == _NS - 1)
    def _():
        pltpu.sync_copy(acc.at[pl.ds(_NS * _NPT, _NREM)],
                        out_hbm.at[cid].at[pl.ds(_NS * _NPT, _NREM)])


@functools.cache
def _sc_scatter_fn():
    return pl.kernel(
        _sc_scatter_body,
        out_type=jax.ShapeDtypeStruct((_NC, _N, _H), jnp.float32),
        mesh=_sc_mesh(),
        scratch_types=[
            pltpu.VMEM((2, _CH), jnp.int32),
            pltpu.VMEM((2, _CH, _H), jnp.float32),
            pltpu.VMEM_SHARED((_NPAD, _H), jnp.float32),
        ] + [pltpu.SemaphoreType.DMA] * 4)


def _sc_scatter(e_new, dst_s, zeros_nh):
    return _sc_scatter_fn()(e_new, dst_s, zeros_nh)


def kernel(graph_x, edge_index, edge_attr, velocity_sequence_noise,
           enc_node, enc_edge, mp_edge, mp_node, dec, norm_stats):
    del velocity_sequence_noise
    node_mean, node_std, edge_mean, edge_std, out_mean, out_std = norm_stats
    f32 = jnp.float32
    r1 = lambda a: a.reshape(1, -1).astype(f32)

    nw1 = enc_node[0] / node_std[:, None]
    nb1 = r1(enc_node[1] - (node_mean / node_std) @ enc_node[0])
    ew1 = enc_edge[0] / edge_std[:, None]
    eb1 = r1(enc_edge[1] - (edge_mean / edge_std) @ enc_edge[0])

    npad = _EP - _E
    pad_iota = jnp.arange(npad, dtype=jnp.int32)
    src_g = jnp.concatenate(
        [edge_index[0].astype(jnp.int32), pad_iota % _N])
    dst = edge_index[1].astype(jnp.int32)
    dst_g = jnp.concatenate([dst, pad_iota % _N])
    dst_s = jnp.concatenate([dst, _N + pad_iota % (_NPAD - _N)])
    edge_attr_p = jnp.concatenate(
        [edge_attr, jnp.zeros((npad, edge_attr.shape[1]), f32)])
    frames = graph_x[:, 1:3]
    zeros_nh = jnp.zeros((_N, _H), f32)

    def edge_w(i):
        we1 = mp_edge[i][0]
        return we1[:_H], we1[_H:2 * _H], we1[2 * _H:]

    w1e0, w1s0, w1d0 = edge_w(0)
    h_v, p, q = _rows_call(
        _enc_node_body, _N // _NB,
        [graph_x, nw1, nb1, enc_node[2], r1(enc_node[3]), enc_node[4],
         r1(enc_node[5]), r1(enc_node[6]), r1(enc_node[7]), w1s0, w1d0],
        [3] + [None] * 10, [(_N, _H)] * 3, [_H] * 3, _NB)

    d1, db1, d2, db2, d3, db3 = dec
    d3f = d3 * out_std[None, :]
    db3f = r1(db3 * out_std + out_mean)

    h_e = None
    out = None
    for i in range(_MP):
        w1e, _, _ = edge_w(i)
        _, wb1, we2, wb2, we3, wb3, wg, wbe = mp_edge[i]

        r = _sc_gather(p, q, src_g, dst_g)

        if i == 0:
            e_new = _rows_call(
                _enc_edge_mlp_body, _EP // _EB,
                [edge_attr_p, ew1, eb1, enc_edge[2], r1(enc_edge[3]),
                 enc_edge[4], r1(enc_edge[5]), r1(enc_edge[6]),
                 r1(enc_edge[7]), r, w1e, r1(wb1), we2, r1(wb2), we3,
                 r1(wb3), r1(wg), r1(wbe)],
                [3] + [None] * 8 + [_H] + [None] * 8, [(_EP, _H)], [_H], _EB)
        else:
            e_new = _rows_call(
                _edge_mlp_body, _EP // _EB,
                [h_e, r, w1e, r1(wb1), we2, r1(wb2), we3, r1(wb3), r1(wg),
                 r1(wbe)],
                [_H, _H] + [None] * 8, [(_EP, _H)], [_H], _EB)

        part = _sc_scatter(e_new, dst_s, zeros_nh)

        wn1, nb1_, wn2, nb2_, wn3, nb3_, ng_, nbe_ = mp_node[i]
        wv, wa = wn1[:_H], wn1[_H:]
        if i < _MP - 1:
            _, w1sn, w1dn = edge_w(i + 1)
            h_v, p, q = _rows_call(
                _node_mlp_pq_body, _N // _NB,
                [h_v, part, wv, wa, r1(nb1_), wn2, r1(nb2_), wn3, r1(nb3_),
                 r1(ng_), r1(nbe_), w1sn, w1dn],
                [_H, (_NC, _H)] + [None] * 11, [(_N, _H)] * 3, [_H] * 3, _NB)
        else:
            out = _rows_call(
                _node_mlp_dec_body, _N // _NB,
                [h_v, part, wv, wa, r1(nb1_), wn2, r1(nb2_), wn3, r1(nb3_),
                 r1(ng_), r1(nbe_), frames, d1, r1(db1), d2, r1(db2), d3f,
                 db3f],
                [_H, (_NC, _H)] + [None] * 9 + [2] + [None] * 6,
                [(_N, 2)], [2], _NB)
        h_e = e_new
    return out


# speedup vs baseline: 1.3572x; 1.0282x over previous
import functools

import jax
import jax.numpy as jnp
from jax import lax
from jax.experimental import pallas as pl
from jax.experimental.pallas import tpu as pltpu
from jax.experimental.pallas import tpu_sc as plsc

_N = 10000
_E = 160000
_H = 128
_MP = 3

_NC = 2
_NS = 16
_NW = _NC * _NS

_CH = 128
_EP = 163840
_JPW = _EP // (_NW * _CH)
_NPAD = 10016
_NPT = 624
_NREM = _N - _NS * _NPT

_NB = 5000
_EB = 8192


@functools.cache
def _sc_mesh():
    return plsc.VectorSubcoreMesh(
        core_axis_name="c", subcore_axis_name="s",
        num_cores=_NC, num_subcores=_NS)


def _f32dot(a, b):
    return jnp.dot(a, b, preferred_element_type=jnp.float32)


def _ln(h, g, b):
    mu = jnp.mean(h, axis=-1, keepdims=True)
    d = h - mu
    var = jnp.mean(d * d, axis=-1, keepdims=True)
    return d * lax.rsqrt(var + 1e-5) * g + b


def _fullspec(shape):
    n = len(shape)
    return pl.BlockSpec(shape, lambda i, _n=n: (0,) * _n)


def _rowspec(blk, d):
    return pl.BlockSpec((blk, d), lambda i: (i, 0))


def _enc_node_body(x_ref, w1_ref, b1_ref, w2_ref, b2_ref, w3_ref, b3_ref,
                   g_ref, be_ref, ws_ref, wd_ref, o_ref, p_ref, q_ref):
    x = x_ref[...]
    t = x[:, 0:1].astype(jnp.int32)
    oh = (lax.broadcasted_iota(jnp.int32, (_NB, 9), 1) == t).astype(jnp.float32)
    feats = jnp.concatenate([x[:, 1:3], oh], axis=-1)
    h = jnp.maximum(_f32dot(feats, w1_ref[...]) + b1_ref[...], 0.0)
    h = jnp.maximum(_f32dot(h, w2_ref[...]) + b2_ref[...], 0.0)
    h = _f32dot(h, w3_ref[...]) + b3_ref[...]
    hv = _ln(h, g_ref[...], be_ref[...])
    o_ref[...] = hv
    p_ref[...] = _f32dot(hv, ws_ref[...])
    q_ref[...] = _f32dot(hv, wd_ref[...])


def _enc_edge_body(x_ref, w1_ref, b1_ref, w2_ref, b2_ref, w3_ref, b3_ref,
                   g_ref, be_ref, o_ref):
    h = jnp.maximum(_f32dot(x_ref[...], w1_ref[...]) + b1_ref[...], 0.0)
    h = jnp.maximum(_f32dot(h, w2_ref[...]) + b2_ref[...], 0.0)
    h = _f32dot(h, w3_ref[...]) + b3_ref[...]
    o_ref[...] = _ln(h, g_ref[...], be_ref[...])


def _enc_edge_mlp_body(x_ref, a1_ref, c1_ref, a2_ref, c2_ref, a3_ref, c3_ref,
                       ag_ref, abe_ref, r_ref, w1_ref, b1_ref, w2_ref, b2_ref,
                       w3_ref, b3_ref, g_ref, be_ref, o_ref):
    h = jnp.maximum(_f32dot(x_ref[...], a1_ref[...]) + c1_ref[...], 0.0)
    h = jnp.maximum(_f32dot(h, a2_ref[...]) + c2_ref[...], 0.0)
    h = _f32dot(h, a3_ref[...]) + c3_ref[...]
    he = _ln(h, ag_ref[...], abe_ref[...])
    h = jnp.maximum(_f32dot(he, w1_ref[...]) + r_ref[...] + b1_ref[...], 0.0)
    h = jnp.maximum(_f32dot(h, w2_ref[...]) + b2_ref[...], 0.0)
    h = _f32dot(h, w3_ref[...]) + b3_ref[...]
    o_ref[...] = _ln(h, g_ref[...], be_ref[...]) + he


def _edge_mlp_body(he_ref, r_ref, w1_ref, b1_ref, w2_ref, b2_ref,
                   w3_ref, b3_ref, g_ref, be_ref, o_ref):
    he = he_ref[...]
    h = jnp.maximum(_f32dot(he, w1_ref[...]) + r_ref[...] + b1_ref[...], 0.0)
    h = jnp.maximum(_f32dot(h, w2_ref[...]) + b2_ref[...], 0.0)
    h = _f32dot(h, w3_ref[...]) + b3_ref[...]
    o_ref[...] = _ln(h, g_ref[...], be_ref[...]) + he


def _node_mlp_pq_body(hv_ref, part_ref, wv_ref, wa_ref, b1_ref, w2_ref,
                      b2_ref, w3_ref, b3_ref, g_ref, be_ref, ws_ref, wd_ref,
                      o_ref, p_ref, q_ref):
    hv = hv_ref[...]
    agg = part_ref[0] + part_ref[1]
    h = jnp.maximum(_f32dot(hv, wv_ref[...]) + _f32dot(agg, wa_ref[...])
                    + b1_ref[...], 0.0)
    h = jnp.maximum(_f32dot(h, w2_ref[...]) + b2_ref[...], 0.0)
    h = _f32dot(h, w3_ref[...]) + b3_ref[...]
    hv = _ln(h, g_ref[...], be_ref[...]) + hv
    o_ref[...] = hv
    p_ref[...] = _f32dot(hv, ws_ref[...])
    q_ref[...] = _f32dot(hv, wd_ref[...])


def _node_mlp_dec_body(hv_ref, part_ref, wv_ref, wa_ref, b1_ref, w2_ref,
                       b2_ref, w3_ref, b3_ref, g_ref, be_ref, fr_ref,
                       d1_ref, e1_ref, d2_ref, e2_ref, d3_ref, e3_ref,
                       o_ref):
    hv = hv_ref[...]
    agg = part_ref[0] + part_ref[1]
    h = jnp.maximum(_f32dot(hv, wv_ref[...]) + _f32dot(agg, wa_ref[...])
                    + b1_ref[...], 0.0)
    h = jnp.maximum(_f32dot(h, w2_ref[...]) + b2_ref[...], 0.0)
    h = _f32dot(h, w3_ref[...]) + b3_ref[...]
    hv = _ln(h, g_ref[...], be_ref[...]) + hv
    h = jnp.maximum(_f32dot(hv, d1_ref[...]) + e1_ref[...], 0.0)
    h = jnp.maximum(_f32dot(h, d2_ref[...]) + e2_ref[...], 0.0)
    o_ref[...] = fr_ref[...] + _f32dot(h, d3_ref[...]) + e3_ref[...]


def _rows_call(body, grid, in_arrays, in_blocked_d, out_shapes, out_d, blk):
    in_specs = []
    for a, d in zip(in_arrays, in_blocked_d):
        if d is None:
            in_specs.append(_fullspec(a.shape))
        elif isinstance(d, tuple):
            in_specs.append(pl.BlockSpec((d[0], blk, d[1]),
                                         lambda i: (0, i, 0)))
        else:
            in_specs.append(_rowspec(blk, d))
    out_specs = [_rowspec(blk, d) for d in out_d]
    out_shape = [jax.ShapeDtypeStruct(s, jnp.float32) for s in out_shapes]
    if len(out_shape) == 1:
        out_shape, out_specs = out_shape[0], out_specs[0]
    return pl.pallas_call(
        body, grid=(grid,), in_specs=in_specs, out_specs=out_specs,
        out_shape=out_shape)(*in_arrays)


def _sc_gather_body(p_hbm, q_hbm, src_hbm, dst_hbm, r_hbm,
                    sidx, didx, pbuf, qbuf,
                    isem0, isem1, gsem0, gsem1, wsem0, wsem1):
    cid = lax.axis_index("c")
    sid = lax.axis_index("s")
    wid = sid * _NC + cid
    isems = (isem0, isem1)
    gsems = (gsem0, gsem1)
    wsems = (wsem0, wsem1)

    def off(g):
        return (g * _NW + wid) * _CH

    def idx_start(g, s):
        pltpu.make_async_copy(src_hbm.at[pl.ds(off(g), _CH)],
                              sidx.at[s], isems[s]).start()
        pltpu.make_async_copy(dst_hbm.at[pl.ds(off(g), _CH)],
                              didx.at[s], isems[s]).start()

    def idx_wait(s):
        pltpu.make_async_copy(src_hbm.at[pl.ds(0, _CH)],
                              sidx.at[s], isems[s]).wait()
        pltpu.make_async_copy(dst_hbm.at[pl.ds(0, _CH)],
                              didx.at[s], isems[s]).wait()

    def gather_start(s):
        pltpu.make_async_copy(p_hbm.at[sidx.at[s]], pbuf.at[s],
                              gsems[s]).start()
        pltpu.make_async_copy(q_hbm.at[didx.at[s]], qbuf.at[s],
                              gsems[s]).start()

    def gather_wait(s):
        pltpu.make_async_copy(p_hbm.at[sidx.at[s]], pbuf.at[s],
                              gsems[s]).wait()
        pltpu.make_async_copy(q_hbm.at[didx.at[s]], qbuf.at[s],
                              gsems[s]).wait()

    def write_start(g, s):
        pltpu.make_async_copy(pbuf.at[s], r_hbm.at[pl.ds(off(g), _CH)],
                              wsems[s]).start()

    def write_wait(s):
        pltpu.make_async_copy(pbuf.at[s], r_hbm.at[pl.ds(0, _CH)],
                              wsems[s]).wait()

    def add_slot(s):
        def addrow(rr, c):
            for cc in range(_H // 16):
                col = pl.ds(cc * 16, 16)
                pbuf[s, rr, col] = pbuf[s, rr, col] + qbuf[s, rr, col]
            return c
        lax.fori_loop(0, _CH, addrow, 0)

    idx_start(0, 0)
    idx_wait(0)
    gather_start(0)
    idx_start(1, 1)

    def body(jj, carry):
        g0 = jj * 2
        gather_wait(0)

        @pl.when(jj < _JPW // 2 - 1)
        def _():
            idx_start(g0 + 2, 0)

        @pl.when(jj > 0)
        def _():
            write_wait(1)

        idx_wait(1)
        gather_start(1)
        add_slot(0)
        write_start(g0, 0)

        gather_wait(1)

        @pl.when(jj < _JPW // 2 - 1)
        def _():
            idx_start(g0 + 3, 1)

        write_wait(0)

        @pl.when(jj < _JPW // 2 - 1)
        def _():
            idx_wait(0)
            gather_start(0)

        add_slot(1)
        write_start(g0 + 1, 1)
        return carry

    lax.fori_loop(0, _JPW // 2, body, 0)
    write_wait(1)


@functools.cache
def _sc_gather_fn():
    return pl.kernel(
        _sc_gather_body,
        out_type=jax.ShapeDtypeStruct((_EP, _H), jnp.float32),
        mesh=_sc_mesh(),
        scratch_types=[
            pltpu.VMEM((2, _CH), jnp.int32),
            pltpu.VMEM((2, _CH), jnp.int32),
            pltpu.VMEM((2, _CH, _H), jnp.float32),
            pltpu.VMEM((2, _CH, _H), jnp.float32),
        ] + [pltpu.SemaphoreType.DMA] * 6)


def _sc_gather(p, q, src_g, dst_g):
    return _sc_gather_fn()(p, q, src_g, dst_g)


def _sc_scatter_body(e_hbm, dst_hbm, z_hbm, out_hbm,
                     didx, ebuf, acc, lsem0, lsem1, ssem0, ssem1):
    cid = lax.axis_index("c")
    sid = lax.axis_index("s")
    wid = sid * _NC + cid
    lsems = (lsem0, lsem1)
    ssems = (ssem0, ssem1)

    def off(g):
        return (g * _NW + wid) * _CH

    def load_start(g, s):
        pltpu.make_async_copy(dst_hbm.at[pl.ds(off(g), _CH)],
                              didx.at[s], lsems[s]).start()
        pltpu.make_async_copy(e_hbm.at[pl.ds(off(g), _CH)],
                              ebuf.at[s], lsems[s]).start()

    def load_wait(s):
        pltpu.make_async_copy(dst_hbm.at[pl.ds(0, _CH)],
                              didx.at[s], lsems[s]).wait()
        pltpu.make_async_copy(e_hbm.at[pl.ds(0, _CH)],
                              ebuf.at[s], lsems[s]).wait()

    def scat_start(s):
        pltpu.make_async_copy(ebuf.at[s], acc.at[didx.at[s]],
                              ssems[s]).start(add=True)

    def scat_wait(s):
        pltpu.make_async_copy(ebuf.at[s], acc.at[didx.at[s]],
                              ssems[s]).wait()

    pltpu.sync_copy(z_hbm.at[pl.ds(sid * _NPT, _NPT)],
                    acc.at[pl.ds(sid * _NPT, _NPT)])

    @pl.when(sid == _NS - 1)
    def _():
        pltpu.sync_copy(z_hbm.at[pl.ds(_NS * _NPT, _NREM)],
                        acc.at[pl.ds(_NS * _NPT, _NREM)])

    plsc.subcore_barrier()

    load_start(0, 0)

    def body(jj, carry):
        g0 = jj * 2
        load_wait(0)
        scat_start(0)

        @pl.when(jj > 0)
        def _():
            scat_wait(1)

        load_start(g0 + 1, 1)

        load_wait(1)
        scat_start(1)
        scat_wait(0)

        @pl.when(jj < _JPW // 2 - 1)
        def _():
            load_start(g0 + 2, 0)

        return carry

    lax.fori_loop(0, _JPW // 2, body, 0)
    scat_wait(1)
    plsc.subcore_barrier()
    pltpu.sync_copy(acc.at[pl.ds(sid * _NPT, _NPT)],
                    out_hbm.at[cid].at[pl.ds(sid * _NPT, _NPT)])

    @pl.when(sid == _NS - 1)
    def _():
        pltpu.sync_copy(acc.at[pl.ds(_NS * _NPT, _NREM)],
                        out_hbm.at[cid].at[pl.ds(_NS * _NPT, _NREM)])


@functools.cache
def _sc_scatter_fn():
    return pl.kernel(
        _sc_scatter_body,
        out_type=jax.ShapeDtypeStruct((_NC, _N, _H), jnp.float32),
        mesh=_sc_mesh(),
        scratch_types=[
            pltpu.VMEM((2, _CH), jnp.int32),
            pltpu.VMEM((2, _CH, _H), jnp.float32),
            pltpu.VMEM_SHARED((_NPAD, _H), jnp.float32),
        ] + [pltpu.SemaphoreType.DMA] * 4)


def _sc_scatter(e_new, dst_s, zeros_nh):
    return _sc_scatter_fn()(e_new, dst_s, zeros_nh)


def kernel(graph_x, edge_index, edge_attr, velocity_sequence_noise,
           enc_node, enc_edge, mp_edge, mp_node, dec, norm_stats):
    del velocity_sequence_noise
    node_mean, node_std, edge_mean, edge_std, out_mean, out_std = norm_stats
    f32 = jnp.float32
    r1 = lambda a: a.reshape(1, -1).astype(f32)

    nw1 = enc_node[0] / node_std[:, None]
    nb1 = r1(enc_node[1] - (node_mean / node_std) @ enc_node[0])
    ew1 = enc_edge[0] / edge_std[:, None]
    eb1 = r1(enc_edge[1] - (edge_mean / edge_std) @ enc_edge[0])

    npad = _EP - _E
    pad_iota = jnp.arange(npad, dtype=jnp.int32)
    src_g = jnp.concatenate(
        [edge_index[0].astype(jnp.int32), pad_iota % _N])
    dst = edge_index[1].astype(jnp.int32)
    dst_g = jnp.concatenate([dst, pad_iota % _N])
    dst_s = jnp.concatenate([dst, _N + pad_iota % (_NPAD - _N)])
    edge_attr_p = jnp.concatenate(
        [edge_attr, jnp.zeros((npad, edge_attr.shape[1]), f32)])
    frames = graph_x[:, 1:3]
    zeros_nh = jnp.zeros((_N, _H), f32)

    def edge_w(i):
        we1 = mp_edge[i][0]
        return we1[:_H], we1[_H:2 * _H], we1[2 * _H:]

    w1e0, w1s0, w1d0 = edge_w(0)
    h_v, p, q = _rows_call(
        _enc_node_body, _N // _NB,
        [graph_x, nw1, nb1, enc_node[2], r1(enc_node[3]), enc_node[4],
         r1(enc_node[5]), r1(enc_node[6]), r1(enc_node[7]), w1s0, w1d0],
        [3] + [None] * 10, [(_N, _H)] * 3, [_H] * 3, _NB)

    d1, db1, d2, db2, d3, db3 = dec
    d3f = d3 * out_std[None, :]
    db3f = r1(db3 * out_std + out_mean)

    h_e = None
    out = None
    for i in range(_MP):
        w1e, _, _ = edge_w(i)
        _, wb1, we2, wb2, we3, wb3, wg, wbe = mp_edge[i]

        r = _sc_gather(p, q, src_g, dst_g)

        if i == 0:
            e_new = _rows_call(
                _enc_edge_mlp_body, _EP // _EB,
                [edge_attr_p, ew1, eb1, enc_edge[2], r1(enc_edge[3]),
                 enc_edge[4], r1(enc_edge[5]), r1(enc_edge[6]),
                 r1(enc_edge[7]), r, w1e, r1(wb1), we2, r1(wb2), we3,
                 r1(wb3), r1(wg), r1(wbe)],
                [3] + [None] * 8 + [_H] + [None] * 8, [(_EP, _H)], [_H], _EB)
        else:
            e_new = _rows_call(
                _edge_mlp_body, _EP // _EB,
                [h_e, r, w1e, r1(wb1), we2, r1(wb2), we3, r1(wb3), r1(wg),
                 r1(wbe)],
                [_H, _H] + [None] * 8, [(_EP, _H)], [_H], _EB)

        part = _sc_scatter(e_new, dst_s, zeros_nh)

        wn1, nb1_, wn2, nb2_, wn3, nb3_, ng_, nbe_ = mp_node[i]
        wv, wa = wn1[:_H], wn1[_H:]
        if i < _MP - 1:
            _, w1sn, w1dn = edge_w(i + 1)
            h_v, p, q = _rows_call(
                _node_mlp_pq_body, _N // _NB,
                [h_v, part, wv, wa, r1(nb1_), wn2, r1(nb2_), wn3, r1(nb3_),
                 r1(ng_), r1(nbe_), w1sn, w1dn],
                [_H, (_NC, _H)] + [None] * 11, [(_N, _H)] * 3, [_H] * 3, _NB)
        else:
            out = _rows_call(
                _node_mlp_dec_body, _N // _NB,
                [h_v, part, wv, wa, r1(nb1_), wn2, r1(nb2_), wn3, r1(nb3_),
                 r1(ng_), r1(nbe_), frames, d1, r1(db1), d2, r1(db2), d3f,
                 db3f],
                [_H, (_NC, _H)] + [None] * 9 + [2] + [None] * 6,
                [(_N, 2)], [2], _NB)
        h_e = e_new
    return out


# speedup vs baseline: 1.3935x; 1.0267x over previous
import functools

import jax
import jax.numpy as jnp
from jax import lax
from jax.experimental import pallas as pl
from jax.experimental.pallas import tpu as pltpu
from jax.experimental.pallas import tpu_sc as plsc

_N = 10000
_E = 160000
_H = 128
_MP = 3

_NC = 2
_NS = 16
_NW = _NC * _NS

_CH = 128
_EP = 163840
_EH = _EP // 2
_JPW = _EH // (_NW * _CH)
_NPAD = 10016
_NPT = 624
_NREM = _N - _NS * _NPT

_NB = 5000
_EB = 8192


@functools.cache
def _sc_mesh():
    return plsc.VectorSubcoreMesh(
        core_axis_name="c", subcore_axis_name="s",
        num_cores=_NC, num_subcores=_NS)


def _f32dot(a, b):
    return jnp.dot(a, b, preferred_element_type=jnp.float32)


def _ln(h, g, b):
    mu = jnp.mean(h, axis=-1, keepdims=True)
    d = h - mu
    var = jnp.mean(d * d, axis=-1, keepdims=True)
    return d * lax.rsqrt(var + 1e-5) * g + b


def _fullspec(shape):
    n = len(shape)
    return pl.BlockSpec(shape, lambda i, _n=n: (0,) * _n)


def _rowspec(blk, d):
    return pl.BlockSpec((blk, d), lambda i: (i, 0))


def _enc_node_body(x_ref, w1_ref, b1_ref, w2_ref, b2_ref, w3_ref, b3_ref,
                   g_ref, be_ref, ws_ref, wd_ref, o_ref, p_ref, q_ref):
    x = x_ref[...]
    t = x[:, 0:1].astype(jnp.int32)
    oh = (lax.broadcasted_iota(jnp.int32, (_NB, 9), 1) == t).astype(jnp.float32)
    feats = jnp.concatenate([x[:, 1:3], oh], axis=-1)
    h = jnp.maximum(_f32dot(feats, w1_ref[...]) + b1_ref[...], 0.0)
    h = jnp.maximum(_f32dot(h, w2_ref[...]) + b2_ref[...], 0.0)
    h = _f32dot(h, w3_ref[...]) + b3_ref[...]
    hv = _ln(h, g_ref[...], be_ref[...])
    o_ref[...] = hv
    p_ref[...] = _f32dot(hv, ws_ref[...])
    q_ref[...] = _f32dot(hv, wd_ref[...])


def _enc_edge_body(x_ref, w1_ref, b1_ref, w2_ref, b2_ref, w3_ref, b3_ref,
                   g_ref, be_ref, o_ref):
    h = jnp.maximum(_f32dot(x_ref[...], w1_ref[...]) + b1_ref[...], 0.0)
    h = jnp.maximum(_f32dot(h, w2_ref[...]) + b2_ref[...], 0.0)
    h = _f32dot(h, w3_ref[...]) + b3_ref[...]
    o_ref[...] = _ln(h, g_ref[...], be_ref[...])


def _enc_edge_mlp_body(x_ref, a1_ref, c1_ref, a2_ref, c2_ref, a3_ref, c3_ref,
                       ag_ref, abe_ref, r_ref, w1_ref, b1_ref, w2_ref, b2_ref,
                       w3_ref, b3_ref, g_ref, be_ref, o_ref):
    h = jnp.maximum(_f32dot(x_ref[...], a1_ref[...]) + c1_ref[...], 0.0)
    h = jnp.maximum(_f32dot(h, a2_ref[...]) + c2_ref[...], 0.0)
    h = _f32dot(h, a3_ref[...]) + c3_ref[...]
    he = _ln(h, ag_ref[...], abe_ref[...])
    h = jnp.maximum(_f32dot(he, w1_ref[...]) + r_ref[...] + b1_ref[...], 0.0)
    h = jnp.maximum(_f32dot(h, w2_ref[...]) + b2_ref[...], 0.0)
    h = _f32dot(h, w3_ref[...]) + b3_ref[...]
    o_ref[...] = _ln(h, g_ref[...], be_ref[...]) + he


def _edge_mlp_body(he_ref, r_ref, w1_ref, b1_ref, w2_ref, b2_ref,
                   w3_ref, b3_ref, g_ref, be_ref, o_ref):
    he = he_ref[...]
    h = jnp.maximum(_f32dot(he, w1_ref[...]) + r_ref[...] + b1_ref[...], 0.0)
    h = jnp.maximum(_f32dot(h, w2_ref[...]) + b2_ref[...], 0.0)
    h = _f32dot(h, w3_ref[...]) + b3_ref[...]
    o_ref[...] = _ln(h, g_ref[...], be_ref[...]) + he


def _node_mlp_pq_body(hv_ref, pa_ref, pb_ref, wv_ref, wa_ref, b1_ref, w2_ref,
                      b2_ref, w3_ref, b3_ref, g_ref, be_ref, ws_ref, wd_ref,
                      o_ref, p_ref, q_ref):
    hv = hv_ref[...]
    agg = (pa_ref[0] + pa_ref[1]) + (pb_ref[0] + pb_ref[1])
    h = jnp.maximum(_f32dot(hv, wv_ref[...]) + _f32dot(agg, wa_ref[...])
                    + b1_ref[...], 0.0)
    h = jnp.maximum(_f32dot(h, w2_ref[...]) + b2_ref[...], 0.0)
    h = _f32dot(h, w3_ref[...]) + b3_ref[...]
    hv = _ln(h, g_ref[...], be_ref[...]) + hv
    o_ref[...] = hv
    p_ref[...] = _f32dot(hv, ws_ref[...])
    q_ref[...] = _f32dot(hv, wd_ref[...])


def _node_mlp_dec_body(hv_ref, pa_ref, pb_ref, wv_ref, wa_ref, b1_ref, w2_ref,
                       b2_ref, w3_ref, b3_ref, g_ref, be_ref, fr_ref,
                       d1_ref, e1_ref, d2_ref, e2_ref, d3_ref, e3_ref,
                       o_ref):
    hv = hv_ref[...]
    agg = (pa_ref[0] + pa_ref[1]) + (pb_ref[0] + pb_ref[1])
    h = jnp.maximum(_f32dot(hv, wv_ref[...]) + _f32dot(agg, wa_ref[...])
                    + b1_ref[...], 0.0)
    h = jnp.maximum(_f32dot(h, w2_ref[...]) + b2_ref[...], 0.0)
    h = _f32dot(h, w3_ref[...]) + b3_ref[...]
    hv = _ln(h, g_ref[...], be_ref[...]) + hv
    h = jnp.maximum(_f32dot(hv, d1_ref[...]) + e1_ref[...], 0.0)
    h = jnp.maximum(_f32dot(h, d2_ref[...]) + e2_ref[...], 0.0)
    o_ref[...] = fr_ref[...] + _f32dot(h, d3_ref[...]) + e3_ref[...]


def _rows_call(body, grid, in_arrays, in_blocked_d, out_shapes, out_d, blk):
    in_specs = []
    for a, d in zip(in_arrays, in_blocked_d):
        if d is None:
            in_specs.append(_fullspec(a.shape))
        elif isinstance(d, tuple):
            in_specs.append(pl.BlockSpec((d[0], blk, d[1]),
                                         lambda i: (0, i, 0)))
        else:
            in_specs.append(_rowspec(blk, d))
    out_specs = [_rowspec(blk, d) for d in out_d]
    out_shape = [jax.ShapeDtypeStruct(s, jnp.float32) for s in out_shapes]
    if len(out_shape) == 1:
        out_shape, out_specs = out_shape[0], out_specs[0]
    return pl.pallas_call(
        body, grid=(grid,), in_specs=in_specs, out_specs=out_specs,
        out_shape=out_shape)(*in_arrays)


def _sc_gather_body(p_hbm, q_hbm, src_hbm, dst_hbm, r_hbm,
                    sidx, didx, pbuf, qbuf,
                    isem0, isem1, gsem0, gsem1, wsem0, wsem1):
    cid = lax.axis_index("c")
    sid = lax.axis_index("s")
    wid = sid * _NC + cid
    isems = (isem0, isem1)
    gsems = (gsem0, gsem1)
    wsems = (wsem0, wsem1)

    def off(g):
        return (g * _NW + wid) * _CH

    def idx_start(g, s):
        pltpu.make_async_copy(src_hbm.at[pl.ds(off(g), _CH)],
                              sidx.at[s], isems[s]).start()
        pltpu.make_async_copy(dst_hbm.at[pl.ds(off(g), _CH)],
                              didx.at[s], isems[s]).start()

    def idx_wait(s):
        pltpu.make_async_copy(src_hbm.at[pl.ds(0, _CH)],
                              sidx.at[s], isems[s]).wait()
        pltpu.make_async_copy(dst_hbm.at[pl.ds(0, _CH)],
                              didx.at[s], isems[s]).wait()

    def gather_start(s):
        pltpu.make_async_copy(p_hbm.at[sidx.at[s]], pbuf.at[s],
                              gsems[s]).start()
        pltpu.make_async_copy(q_hbm.at[didx.at[s]], qbuf.at[s],
                              gsems[s]).start()

    def gather_wait(s):
        pltpu.make_async_copy(p_hbm.at[sidx.at[s]], pbuf.at[s],
                              gsems[s]).wait()
        pltpu.make_async_copy(q_hbm.at[didx.at[s]], qbuf.at[s],
                              gsems[s]).wait()

    def write_start(g, s):
        pltpu.make_async_copy(pbuf.at[s], r_hbm.at[pl.ds(off(g), _CH)],
                              wsems[s]).start()

    def write_wait(s):
        pltpu.make_async_copy(pbuf.at[s], r_hbm.at[pl.ds(0, _CH)],
                              wsems[s]).wait()

    def add_slot(s):
        def addrow(rr, c):
            for cc in range(_H // 16):
                col = pl.ds(cc * 16, 16)
                pbuf[s, rr, col] = pbuf[s, rr, col] + qbuf[s, rr, col]
            return c
        lax.fori_loop(0, _CH, addrow, 0)

    idx_start(0, 0)
    idx_wait(0)
    gather_start(0)
    idx_start(1, 1)

    def body(jj, carry):
        g0 = jj * 2
        gather_wait(0)

        @pl.when(jj < _JPW // 2 - 1)
        def _():
            idx_start(g0 + 2, 0)

        @pl.when(jj > 0)
        def _():
            write_wait(1)

        idx_wait(1)
        gather_start(1)
        add_slot(0)
        write_start(g0, 0)

        gather_wait(1)

        @pl.when(jj < _JPW // 2 - 1)
        def _():
            idx_start(g0 + 3, 1)

        write_wait(0)

        @pl.when(jj < _JPW // 2 - 1)
        def _():
            idx_wait(0)
            gather_start(0)

        add_slot(1)
        write_start(g0 + 1, 1)
        return carry

    lax.fori_loop(0, _JPW // 2, body, 0)
    write_wait(1)


@functools.cache
def _sc_gather_fn():
    return pl.kernel(
        _sc_gather_body,
        out_type=jax.ShapeDtypeStruct((_EH, _H), jnp.float32),
        mesh=_sc_mesh(),
        scratch_types=[
            pltpu.VMEM((2, _CH), jnp.int32),
            pltpu.VMEM((2, _CH), jnp.int32),
            pltpu.VMEM((2, _CH, _H), jnp.float32),
            pltpu.VMEM((2, _CH, _H), jnp.float32),
        ] + [pltpu.SemaphoreType.DMA] * 6)


def _sc_gather(p, q, src_g, dst_g):
    return _sc_gather_fn()(p, q, src_g, dst_g)


def _sc_scatter_body(e_hbm, dst_hbm, z_hbm, out_hbm,
                     didx, ebuf, acc, lsem0, lsem1, ssem0, ssem1):
    cid = lax.axis_index("c")
    sid = lax.axis_index("s")
    wid = sid * _NC + cid
    lsems = (lsem0, lsem1)
    ssems = (ssem0, ssem1)

    def off(g):
        return (g * _NW + wid) * _CH

    def load_start(g, s):
        pltpu.make_async_copy(dst_hbm.at[pl.ds(off(g), _CH)],
                              didx.at[s], lsems[s]).start()
        pltpu.make_async_copy(e_hbm.at[pl.ds(off(g), _CH)],
                              ebuf.at[s], lsems[s]).start()

    def load_wait(s):
        pltpu.make_async_copy(dst_hbm.at[pl.ds(0, _CH)],
                              didx.at[s], lsems[s]).wait()
        pltpu.make_async_copy(e_hbm.at[pl.ds(0, _CH)],
                              ebuf.at[s], lsems[s]).wait()

    def scat_start(s):
        pltpu.make_async_copy(ebuf.at[s], acc.at[didx.at[s]],
                              ssems[s]).start(add=True)

    def scat_wait(s):
        pltpu.make_async_copy(ebuf.at[s], acc.at[didx.at[s]],
                              ssems[s]).wait()

    pltpu.sync_copy(z_hbm.at[pl.ds(sid * _NPT, _NPT)],
                    acc.at[pl.ds(sid * _NPT, _NPT)])

    @pl.when(sid == _NS - 1)
    def _():
        pltpu.sync_copy(z_hbm.at[pl.ds(_NS * _NPT, _NREM)],
                        acc.at[pl.ds(_NS * _NPT, _NREM)])

    plsc.subcore_barrier()

    load_start(0, 0)

    def body(jj, carry):
        g0 = jj * 2
        load_wait(0)
        scat_start(0)

        @pl.when(jj > 0)
        def _():
            scat_wait(1)

        load_start(g0 + 1, 1)

        load_wait(1)
        scat_start(1)
        scat_wait(0)

        @pl.when(jj < _JPW // 2 - 1)
        def _():
            load_start(g0 + 2, 0)

        return carry

    lax.fori_loop(0, _JPW // 2, body, 0)
    scat_wait(1)
    plsc.subcore_barrier()
    pltpu.sync_copy(acc.at[pl.ds(sid * _NPT, _NPT)],
                    out_hbm.at[cid].at[pl.ds(sid * _NPT, _NPT)])

    @pl.when(sid == _NS - 1)
    def _():
        pltpu.sync_copy(acc.at[pl.ds(_NS * _NPT, _NREM)],
                        out_hbm.at[cid].at[pl.ds(_NS * _NPT, _NREM)])


@functools.cache
def _sc_scatter_fn():
    return pl.kernel(
        _sc_scatter_body,
        out_type=jax.ShapeDtypeStruct((_NC, _N, _H), jnp.float32),
        mesh=_sc_mesh(),
        scratch_types=[
            pltpu.VMEM((2, _CH), jnp.int32),
            pltpu.VMEM((2, _CH, _H), jnp.float32),
            pltpu.VMEM_SHARED((_NPAD, _H), jnp.float32),
        ] + [pltpu.SemaphoreType.DMA] * 4)


def _sc_scatter(e_new, dst_s, zeros_nh):
    return _sc_scatter_fn()(e_new, dst_s, zeros_nh)


def kernel(graph_x, edge_index, edge_attr, velocity_sequence_noise,
           enc_node, enc_edge, mp_edge, mp_node, dec, norm_stats):
    del velocity_sequence_noise
    node_mean, node_std, edge_mean, edge_std, out_mean, out_std = norm_stats
    f32 = jnp.float32
    r1 = lambda a: a.reshape(1, -1).astype(f32)

    nw1 = enc_node[0] / node_std[:, None]
    nb1 = r1(enc_node[1] - (node_mean / node_std) @ enc_node[0])
    ew1 = enc_edge[0] / edge_std[:, None]
    eb1 = r1(enc_edge[1] - (edge_mean / edge_std) @ enc_edge[0])

    npad = _EP - _E
    pad_iota = jnp.arange(npad, dtype=jnp.int32)
    src_g = jnp.concatenate(
        [edge_index[0].astype(jnp.int32), pad_iota % _N])
    dst = edge_index[1].astype(jnp.int32)
    dst_g = jnp.concatenate([dst, pad_iota % _N])
    dst_s = jnp.concatenate([dst, _N + pad_iota % (_NPAD - _N)])
    edge_attr_p = jnp.concatenate(
        [edge_attr, jnp.zeros((npad, edge_attr.shape[1]), f32)])
    src_gh = (src_g[:_EH], src_g[_EH:])
    dst_gh = (dst_g[:_EH], dst_g[_EH:])
    dst_sh = (dst_s[:_EH], dst_s[_EH:])
    edge_attr_h = (edge_attr_p[:_EH], edge_attr_p[_EH:])
    frames = graph_x[:, 1:3]
    zeros_nh = jnp.zeros((_N, _H), f32)

    def edge_w(i):
        we1 = mp_edge[i][0]
        return we1[:_H], we1[_H:2 * _H], we1[2 * _H:]

    w1e0, w1s0, w1d0 = edge_w(0)
    h_v, p, q = _rows_call(
        _enc_node_body, _N // _NB,
        [graph_x, nw1, nb1, enc_node[2], r1(enc_node[3]), enc_node[4],
         r1(enc_node[5]), r1(enc_node[6]), r1(enc_node[7]), w1s0, w1d0],
        [3] + [None] * 10, [(_N, _H)] * 3, [_H] * 3, _NB)

    d1, db1, d2, db2, d3, db3 = dec
    d3f = d3 * out_std[None, :]
    db3f = r1(db3 * out_std + out_mean)

    h_e = (None, None)
    out = None
    for i in range(_MP):
        w1e, _, _ = edge_w(i)
        _, wb1, we2, wb2, we3, wb3, wg, wbe = mp_edge[i]

        def edge_mlp(half, r, _i=i, _w=(w1e, wb1, we2, wb2, we3, wb3, wg,
                                        wbe)):
            w1e_, wb1_, we2_, wb2_, we3_, wb3_, wg_, wbe_ = _w
            if _i == 0:
                return _rows_call(
                    _enc_edge_mlp_body, _EH // _EB,
                    [edge_attr_h[half], ew1, eb1, enc_edge[2],
                     r1(enc_edge[3]), enc_edge[4], r1(enc_edge[5]),
                     r1(enc_edge[6]), r1(enc_edge[7]), r, w1e_, r1(wb1_),
                     we2_, r1(wb2_), we3_, r1(wb3_), r1(wg_), r1(wbe_)],
                    [3] + [None] * 8 + [_H] + [None] * 8,
                    [(_EH, _H)], [_H], _EB)
            return _rows_call(
                _edge_mlp_body, _EH // _EB,
                [h_e[half], r, w1e_, r1(wb1_), we2_, r1(wb2_), we3_,
                 r1(wb3_), r1(wg_), r1(wbe_)],
                [_H, _H] + [None] * 8, [(_EH, _H)], [_H], _EB)

        r_a = _sc_gather(p, q, src_gh[0], dst_gh[0])
        e_new_a = edge_mlp(0, r_a)
        r_b = _sc_gather(p, q, src_gh[1], dst_gh[1])
        part_a = _sc_scatter(e_new_a, dst_sh[0], zeros_nh)
        e_new_b = edge_mlp(1, r_b)
        part_b = _sc_scatter(e_new_b, dst_sh[1], zeros_nh)

        wn1, nb1_, wn2, nb2_, wn3, nb3_, ng_, nbe_ = mp_node[i]
        wv, wa = wn1[:_H], wn1[_H:]
        if i < _MP - 1:
            _, w1sn, w1dn = edge_w(i + 1)
            h_v, p, q = _rows_call(
                _node_mlp_pq_body, _N // _NB,
                [h_v, part_a, part_b, wv, wa, r1(nb1_), wn2, r1(nb2_), wn3,
                 r1(nb3_), r1(ng_), r1(nbe_), w1sn, w1dn],
                [_H, (_NC, _H), (_NC, _H)] + [None] * 11,
                [(_N, _H)] * 3, [_H] * 3, _NB)
        else:
            out = _rows_call(
                _node_mlp_dec_body, _N // _NB,
                [h_v, part_a, part_b, wv, wa, r1(nb1_), wn2, r1(nb2_), wn3,
                 r1(nb3_), r1(ng_), r1(nbe_), frames, d1, r1(db1), d2,
                 r1(db2), d3f, db3f],
                [_H, (_NC, _H), (_NC, _H)] + [None] * 9 + [2] + [None] * 6,
                [(_N, 2)], [2], _NB)
        h_e = (e_new_a, e_new_b)
    return out


# speedup vs baseline: 1.3969x; 1.0025x over previous
import functools

import jax
import jax.numpy as jnp
from jax import lax
from jax.experimental import pallas as pl
from jax.experimental.pallas import tpu as pltpu
from jax.experimental.pallas import tpu_sc as plsc

_N = 10000
_E = 160000
_H = 128
_MP = 3

_NC = 2
_NS = 16
_NW = _NC * _NS

_CH = 128
_EP = 163840
_EH = _EP // 2
_JPW = _EH // (_NW * _CH)
_NPAD = 10016
_NPT = 624
_NREM = _N - _NS * _NPT

_NB = 5000
_EB = 10240


@functools.cache
def _sc_mesh():
    return plsc.VectorSubcoreMesh(
        core_axis_name="c", subcore_axis_name="s",
        num_cores=_NC, num_subcores=_NS)


def _f32dot(a, b):
    return jnp.dot(a, b, preferred_element_type=jnp.float32)


def _ln(h, g, b):
    mu = jnp.mean(h, axis=-1, keepdims=True)
    d = h - mu
    var = jnp.mean(d * d, axis=-1, keepdims=True)
    return d * lax.rsqrt(var + 1e-5) * g + b


def _fullspec(shape):
    n = len(shape)
    return pl.BlockSpec(shape, lambda i, _n=n: (0,) * _n)


def _rowspec(blk, d):
    return pl.BlockSpec((blk, d), lambda i: (i, 0))


def _enc_node_body(x_ref, w1_ref, b1_ref, w2_ref, b2_ref, w3_ref, b3_ref,
                   g_ref, be_ref, ws_ref, wd_ref, o_ref, p_ref, q_ref):
    x = x_ref[...]
    t = x[:, 0:1].astype(jnp.int32)
    oh = (lax.broadcasted_iota(jnp.int32, (_NB, 9), 1) == t).astype(jnp.float32)
    feats = jnp.concatenate([x[:, 1:3], oh], axis=-1)
    h = jnp.maximum(_f32dot(feats, w1_ref[...]) + b1_ref[...], 0.0)
    h = jnp.maximum(_f32dot(h, w2_ref[...]) + b2_ref[...], 0.0)
    h = _f32dot(h, w3_ref[...]) + b3_ref[...]
    hv = _ln(h, g_ref[...], be_ref[...])
    o_ref[...] = hv
    p_ref[...] = _f32dot(hv, ws_ref[...])
    q_ref[...] = _f32dot(hv, wd_ref[...])


def _enc_edge_body(x_ref, w1_ref, b1_ref, w2_ref, b2_ref, w3_ref, b3_ref,
                   g_ref, be_ref, o_ref):
    h = jnp.maximum(_f32dot(x_ref[...], w1_ref[...]) + b1_ref[...], 0.0)
    h = jnp.maximum(_f32dot(h, w2_ref[...]) + b2_ref[...], 0.0)
    h = _f32dot(h, w3_ref[...]) + b3_ref[...]
    o_ref[...] = _ln(h, g_ref[...], be_ref[...])


def _enc_edge_mlp_body(x_ref, a1_ref, c1_ref, a2_ref, c2_ref, a3_ref, c3_ref,
                       ag_ref, abe_ref, r_ref, w1_ref, b1_ref, w2_ref, b2_ref,
                       w3_ref, b3_ref, g_ref, be_ref, o_ref):
    h = jnp.maximum(_f32dot(x_ref[...], a1_ref[...]) + c1_ref[...], 0.0)
    h = jnp.maximum(_f32dot(h, a2_ref[...]) + c2_ref[...], 0.0)
    h = _f32dot(h, a3_ref[...]) + c3_ref[...]
    he = _ln(h, ag_ref[...], abe_ref[...])
    h = jnp.maximum(_f32dot(he, w1_ref[...]) + r_ref[...] + b1_ref[...], 0.0)
    h = jnp.maximum(_f32dot(h, w2_ref[...]) + b2_ref[...], 0.0)
    h = _f32dot(h, w3_ref[...]) + b3_ref[...]
    o_ref[...] = _ln(h, g_ref[...], be_ref[...]) + he


def _edge_mlp_body(he_ref, r_ref, w1_ref, b1_ref, w2_ref, b2_ref,
                   w3_ref, b3_ref, g_ref, be_ref, o_ref):
    he = he_ref[...]
    h = jnp.maximum(_f32dot(he, w1_ref[...]) + r_ref[...] + b1_ref[...], 0.0)
    h = jnp.maximum(_f32dot(h, w2_ref[...]) + b2_ref[...], 0.0)
    h = _f32dot(h, w3_ref[...]) + b3_ref[...]
    o_ref[...] = _ln(h, g_ref[...], be_ref[...]) + he


def _node_mlp_pq_body(hv_ref, pa_ref, pb_ref, wv_ref, wa_ref, b1_ref, w2_ref,
                      b2_ref, w3_ref, b3_ref, g_ref, be_ref, ws_ref, wd_ref,
                      o_ref, p_ref, q_ref):
    hv = hv_ref[...]
    agg = (pa_ref[0] + pa_ref[1]) + (pb_ref[0] + pb_ref[1])
    h = jnp.maximum(_f32dot(hv, wv_ref[...]) + _f32dot(agg, wa_ref[...])
                    + b1_ref[...], 0.0)
    h = jnp.maximum(_f32dot(h, w2_ref[...]) + b2_ref[...], 0.0)
    h = _f32dot(h, w3_ref[...]) + b3_ref[...]
    hv = _ln(h, g_ref[...], be_ref[...]) + hv
    o_ref[...] = hv
    p_ref[...] = _f32dot(hv, ws_ref[...])
    q_ref[...] = _f32dot(hv, wd_ref[...])


def _node_mlp_dec_body(hv_ref, pa_ref, pb_ref, wv_ref, wa_ref, b1_ref, w2_ref,
                       b2_ref, w3_ref, b3_ref, g_ref, be_ref, fr_ref,
                       d1_ref, e1_ref, d2_ref, e2_ref, d3_ref, e3_ref,
                       o_ref):
    hv = hv_ref[...]
    agg = (pa_ref[0] + pa_ref[1]) + (pb_ref[0] + pb_ref[1])
    h = jnp.maximum(_f32dot(hv, wv_ref[...]) + _f32dot(agg, wa_ref[...])
                    + b1_ref[...], 0.0)
    h = jnp.maximum(_f32dot(h, w2_ref[...]) + b2_ref[...], 0.0)
    h = _f32dot(h, w3_ref[...]) + b3_ref[...]
    hv = _ln(h, g_ref[...], be_ref[...]) + hv
    h = jnp.maximum(_f32dot(hv, d1_ref[...]) + e1_ref[...], 0.0)
    h = jnp.maximum(_f32dot(h, d2_ref[...]) + e2_ref[...], 0.0)
    o_ref[...] = fr_ref[...] + _f32dot(h, d3_ref[...]) + e3_ref[...]


def _rows_call(body, grid, in_arrays, in_blocked_d, out_shapes, out_d, blk):
    in_specs = []
    for a, d in zip(in_arrays, in_blocked_d):
        if d is None:
            in_specs.append(_fullspec(a.shape))
        elif isinstance(d, tuple):
            in_specs.append(pl.BlockSpec((d[0], blk, d[1]),
                                         lambda i: (0, i, 0)))
        else:
            in_specs.append(_rowspec(blk, d))
    out_specs = [_rowspec(blk, d) for d in out_d]
    out_shape = [jax.ShapeDtypeStruct(s, jnp.float32) for s in out_shapes]
    if len(out_shape) == 1:
        out_shape, out_specs = out_shape[0], out_specs[0]
    return pl.pallas_call(
        body, grid=(grid,), in_specs=in_specs, out_specs=out_specs,
        out_shape=out_shape)(*in_arrays)


def _sc_gather_body(p_hbm, q_hbm, src_hbm, dst_hbm, r_hbm,
                    sidx, didx, pbuf, qbuf,
                    isem0, isem1, gsem0, gsem1, wsem0, wsem1):
    cid = lax.axis_index("c")
    sid = lax.axis_index("s")
    wid = sid * _NC + cid
    isems = (isem0, isem1)
    gsems = (gsem0, gsem1)
    wsems = (wsem0, wsem1)

    def off(g):
        return (g * _NW + wid) * _CH

    def idx_start(g, s):
        pltpu.make_async_copy(src_hbm.at[pl.ds(off(g), _CH)],
                              sidx.at[s], isems[s]).start()
        pltpu.make_async_copy(dst_hbm.at[pl.ds(off(g), _CH)],
                              didx.at[s], isems[s]).start()

    def idx_wait(s):
        pltpu.make_async_copy(src_hbm.at[pl.ds(0, _CH)],
                              sidx.at[s], isems[s]).wait()
        pltpu.make_async_copy(dst_hbm.at[pl.ds(0, _CH)],
                              didx.at[s], isems[s]).wait()

    def gather_start(s):
        pltpu.make_async_copy(p_hbm.at[sidx.at[s]], pbuf.at[s],
                              gsems[s]).start()
        pltpu.make_async_copy(q_hbm.at[didx.at[s]], qbuf.at[s],
                              gsems[s]).start()

    def gather_wait(s):
        pltpu.make_async_copy(p_hbm.at[sidx.at[s]], pbuf.at[s],
                              gsems[s]).wait()
        pltpu.make_async_copy(q_hbm.at[didx.at[s]], qbuf.at[s],
                              gsems[s]).wait()

    def write_start(g, s):
        pltpu.make_async_copy(pbuf.at[s], r_hbm.at[pl.ds(off(g), _CH)],
                              wsems[s]).start()

    def write_wait(s):
        pltpu.make_async_copy(pbuf.at[s], r_hbm.at[pl.ds(0, _CH)],
                              wsems[s]).wait()

    def add_slot(s):
        def addrow(rr, c):
            for cc in range(_H // 16):
                col = pl.ds(cc * 16, 16)
                pbuf[s, rr, col] = pbuf[s, rr, col] + qbuf[s, rr, col]
            return c
        lax.fori_loop(0, _CH, addrow, 0)

    idx_start(0, 0)
    idx_wait(0)
    gather_start(0)
    idx_start(1, 1)

    def body(jj, carry):
        g0 = jj * 2
        gather_wait(0)

        @pl.when(jj < _JPW // 2 - 1)
        def _():
            idx_start(g0 + 2, 0)

        @pl.when(jj > 0)
        def _():
            write_wait(1)

        idx_wait(1)
        gather_start(1)
        add_slot(0)
        write_start(g0, 0)

        gather_wait(1)

        @pl.when(jj < _JPW // 2 - 1)
        def _():
            idx_start(g0 + 3, 1)

        write_wait(0)

        @pl.when(jj < _JPW // 2 - 1)
        def _():
            idx_wait(0)
            gather_start(0)

        add_slot(1)
        write_start(g0 + 1, 1)
        return carry

    lax.fori_loop(0, _JPW // 2, body, 0)
    write_wait(1)


@functools.cache
def _sc_gather_fn():
    return pl.kernel(
        _sc_gather_body,
        out_type=jax.ShapeDtypeStruct((_EH, _H), jnp.float32),
        mesh=_sc_mesh(),
        scratch_types=[
            pltpu.VMEM((2, _CH), jnp.int32),
            pltpu.VMEM((2, _CH), jnp.int32),
            pltpu.VMEM((2, _CH, _H), jnp.float32),
            pltpu.VMEM((2, _CH, _H), jnp.float32),
        ] + [pltpu.SemaphoreType.DMA] * 6)


def _sc_gather(p, q, src_g, dst_g):
    return _sc_gather_fn()(p, q, src_g, dst_g)


def _sc_scatter_body(e_hbm, dst_hbm, z_hbm, out_hbm,
                     didx, ebuf, acc, lsem0, lsem1, ssem0, ssem1):
    cid = lax.axis_index("c")
    sid = lax.axis_index("s")
    wid = sid * _NC + cid
    lsems = (lsem0, lsem1)
    ssems = (ssem0, ssem1)

    def off(g):
        return (g * _NW + wid) * _CH

    def load_start(g, s):
        pltpu.make_async_copy(dst_hbm.at[pl.ds(off(g), _CH)],
                              didx.at[s], lsems[s]).start()
        pltpu.make_async_copy(e_hbm.at[pl.ds(off(g), _CH)],
                              ebuf.at[s], lsems[s]).start()

    def load_wait(s):
        pltpu.make_async_copy(dst_hbm.at[pl.ds(0, _CH)],
                              didx.at[s], lsems[s]).wait()
        pltpu.make_async_copy(e_hbm.at[pl.ds(0, _CH)],
                              ebuf.at[s], lsems[s]).wait()

    def scat_start(s):
        pltpu.make_async_copy(ebuf.at[s], acc.at[didx.at[s]],
                              ssems[s]).start(add=True)

    def scat_wait(s):
        pltpu.make_async_copy(ebuf.at[s], acc.at[didx.at[s]],
                              ssems[s]).wait()

    pltpu.sync_copy(z_hbm.at[pl.ds(sid * _NPT, _NPT)],
                    acc.at[pl.ds(sid * _NPT, _NPT)])

    @pl.when(sid == _NS - 1)
    def _():
        pltpu.sync_copy(z_hbm.at[pl.ds(_NS * _NPT, _NREM)],
                        acc.at[pl.ds(_NS * _NPT, _NREM)])

    plsc.subcore_barrier()

    load_start(0, 0)

    def body(jj, carry):
        g0 = jj * 2
        load_wait(0)
        scat_start(0)

        @pl.when(jj > 0)
        def _():
            scat_wait(1)

        load_start(g0 + 1, 1)

        load_wait(1)
        scat_start(1)
        scat_wait(0)

        @pl.when(jj < _JPW // 2 - 1)
        def _():
            load_start(g0 + 2, 0)

        return carry

    lax.fori_loop(0, _JPW // 2, body, 0)
    scat_wait(1)
    plsc.subcore_barrier()
    pltpu.sync_copy(acc.at[pl.ds(sid * _NPT, _NPT)],
                    out_hbm.at[cid].at[pl.ds(sid * _NPT, _NPT)])

    @pl.when(sid == _NS - 1)
    def _():
        pltpu.sync_copy(acc.at[pl.ds(_NS * _NPT, _NREM)],
                        out_hbm.at[cid].at[pl.ds(_NS * _NPT, _NREM)])


@functools.cache
def _sc_scatter_fn():
    return pl.kernel(
        _sc_scatter_body,
        out_type=jax.ShapeDtypeStruct((_NC, _N, _H), jnp.float32),
        mesh=_sc_mesh(),
        scratch_types=[
            pltpu.VMEM((2, _CH), jnp.int32),
            pltpu.VMEM((2, _CH, _H), jnp.float32),
            pltpu.VMEM_SHARED((_NPAD, _H), jnp.float32),
        ] + [pltpu.SemaphoreType.DMA] * 4)


def _sc_scatter(e_new, dst_s, zeros_nh):
    return _sc_scatter_fn()(e_new, dst_s, zeros_nh)


def kernel(graph_x, edge_index, edge_attr, velocity_sequence_noise,
           enc_node, enc_edge, mp_edge, mp_node, dec, norm_stats):
    del velocity_sequence_noise
    node_mean, node_std, edge_mean, edge_std, out_mean, out_std = norm_stats
    f32 = jnp.float32
    r1 = lambda a: a.reshape(1, -1).astype(f32)

    nw1 = enc_node[0] / node_std[:, None]
    nb1 = r1(enc_node[1] - (node_mean / node_std) @ enc_node[0])
    ew1 = enc_edge[0] / edge_std[:, None]
    eb1 = r1(enc_edge[1] - (edge_mean / edge_std) @ enc_edge[0])

    npad = _EP - _E
    pad_iota = jnp.arange(npad, dtype=jnp.int32)
    src_g = jnp.concatenate(
        [edge_index[0].astype(jnp.int32), pad_iota % _N])
    dst = edge_index[1].astype(jnp.int32)
    dst_g = jnp.concatenate([dst, pad_iota % _N])
    dst_s = jnp.concatenate([dst, _N + pad_iota % (_NPAD - _N)])
    edge_attr_p = jnp.concatenate(
        [edge_attr, jnp.zeros((npad, edge_attr.shape[1]), f32)])
    src_gh = (src_g[:_EH], src_g[_EH:])
    dst_gh = (dst_g[:_EH], dst_g[_EH:])
    dst_sh = (dst_s[:_EH], dst_s[_EH:])
    edge_attr_h = (edge_attr_p[:_EH], edge_attr_p[_EH:])
    frames = graph_x[:, 1:3]
    zeros_nh = jnp.zeros((_N, _H), f32)

    def edge_w(i):
        we1 = mp_edge[i][0]
        return we1[:_H], we1[_H:2 * _H], we1[2 * _H:]

    w1e0, w1s0, w1d0 = edge_w(0)
    h_v, p, q = _rows_call(
        _enc_node_body, _N // _NB,
        [graph_x, nw1, nb1, enc_node[2], r1(enc_node[3]), enc_node[4],
         r1(enc_node[5]), r1(enc_node[6]), r1(enc_node[7]), w1s0, w1d0],
        [3] + [None] * 10, [(_N, _H)] * 3, [_H] * 3, _NB)

    d1, db1, d2, db2, d3, db3 = dec
    d3f = d3 * out_std[None, :]
    db3f = r1(db3 * out_std + out_mean)

    h_e = (None, None)
    out = None
    for i in range(_MP):
        w1e, _, _ = edge_w(i)
        _, wb1, we2, wb2, we3, wb3, wg, wbe = mp_edge[i]

        def edge_mlp(half, r, _i=i, _w=(w1e, wb1, we2, wb2, we3, wb3, wg,
                                        wbe)):
            w1e_, wb1_, we2_, wb2_, we3_, wb3_, wg_, wbe_ = _w
            if _i == 0:
                return _rows_call(
                    _enc_edge_mlp_body, _EH // _EB,
                    [edge_attr_h[half], ew1, eb1, enc_edge[2],
                     r1(enc_edge[3]), enc_edge[4], r1(enc_edge[5]),
                     r1(enc_edge[6]), r1(enc_edge[7]), r, w1e_, r1(wb1_),
                     we2_, r1(wb2_), we3_, r1(wb3_), r1(wg_), r1(wbe_)],
                    [3] + [None] * 8 + [_H] + [None] * 8,
                    [(_EH, _H)], [_H], _EB)
            return _rows_call(
                _edge_mlp_body, _EH // _EB,
                [h_e[half], r, w1e_, r1(wb1_), we2_, r1(wb2_), we3_,
                 r1(wb3_), r1(wg_), r1(wbe_)],
                [_H, _H] + [None] * 8, [(_EH, _H)], [_H], _EB)

        r_a = _sc_gather(p, q, src_gh[0], dst_gh[0])
        e_new_a = edge_mlp(0, r_a)
        r_b = _sc_gather(p, q, src_gh[1], dst_gh[1])
        part_a = _sc_scatter(e_new_a, dst_sh[0], zeros_nh)
        e_new_b = edge_mlp(1, r_b)
        part_b = _sc_scatter(e_new_b, dst_sh[1], zeros_nh)

        wn1, nb1_, wn2, nb2_, wn3, nb3_, ng_, nbe_ = mp_node[i]
        wv, wa = wn1[:_H], wn1[_H:]
        if i < _MP - 1:
            _, w1sn, w1dn = edge_w(i + 1)
            h_v, p, q = _rows_call(
                _node_mlp_pq_body, _N // _NB,
                [h_v, part_a, part_b, wv, wa, r1(nb1_), wn2, r1(nb2_), wn3,
                 r1(nb3_), r1(ng_), r1(nbe_), w1sn, w1dn],
                [_H, (_NC, _H), (_NC, _H)] + [None] * 11,
                [(_N, _H)] * 3, [_H] * 3, _NB)
        else:
            out = _rows_call(
                _node_mlp_dec_body, _N // _NB,
                [h_v, part_a, part_b, wv, wa, r1(nb1_), wn2, r1(nb2_), wn3,
                 r1(nb3_), r1(ng_), r1(nbe_), frames, d1, r1(db1), d2,
                 r1(db2), d3f, db3f],
                [_H, (_NC, _H), (_NC, _H)] + [None] * 9 + [2] + [None] * 6,
                [(_N, 2)], [2], _NB)
        h_e = (e_new_a, e_new_b)
    return out
